# Initial kernel scaffold; baseline (speedup 1.0000x reference)
import jax
import jax.numpy as jnp
from jax.experimental import pallas as pl


def kernel(text_embedding, x, edge_index, batch, gcn1_W, gcn1_b, gcn2_W, gcn2_b, ln_w, ln_b, W_ih0, W_hh0, b_ih0, b_hh0, W_ih1, W_hh1, b_ih1, b_hh1, W_nt, b_nt, W_cp, b_cp):
    raise NotImplementedError("write your pallas kernel here")


# speedup vs baseline: 9.6866x; 9.6866x over previous
import functools

import jax
import jax.numpy as jnp
from jax import lax
from jax.experimental import pallas as pl
from jax.experimental.pallas import tpu as pltpu
from jax.experimental.pallas import tpu_sc as plsc

_N = 50000
_E = 800000
_B = 256
_NT = 74
_GH = 64
_SH = 128
_MN = 100

_SUB = 128
_EP = 819200
_ROWS = _EP // _SUB
_RPT = _ROWS // 16
_STAGE = 40
_DEG_RPT = _ROWS // 32
_DEG_PAD = 51200
_HN = _N // 2
_SPROWS = 25600
_DUMP = 25088

_BN = 2000
_GRID = _N // _BN

@functools.lru_cache(maxsize=None)
def _sc_mesh():
    return plsc.VectorSubcoreMesh(core_axis_name="c", subcore_axis_name="s",
                                  num_cores=2, num_subcores=16)


def _deg_body(dst_hbm, out_hbm, zbuf, idxbuf, onesbuf, deg_sp):
    c = lax.axis_index("c")
    s = lax.axis_index("s")

    def _zero(i, _):
        zbuf[pl.ds(i * 16, 16)] = jnp.zeros((16,), jnp.float32)
        return _
    lax.fori_loop(0, 3200 // 16, _zero, None)
    pltpu.sync_copy(zbuf, deg_sp.at[pl.ds(s * 3200, 3200)])

    def _ones(i, _):
        onesbuf[pl.ds(i * 16, 16)] = jnp.ones((16,), jnp.float32)
        return _
    lax.fori_loop(0, 8, _ones, None)
    plsc.subcore_barrier()

    w = s * 2 + c
    pltpu.sync_copy(dst_hbm.at[pl.ds(w * _DEG_RPT, _DEG_RPT)], idxbuf)

    def _scat(i, _):
        pltpu.sync_copy(onesbuf, deg_sp.at[idxbuf.at[i]], add=True)
        return _
    lax.fori_loop(0, _DEG_RPT, _scat, None)
    plsc.subcore_barrier()
    pltpu.sync_copy(deg_sp.at[pl.ds(s * 3200, 3200)],
                    out_hbm.at[c, 0, pl.ds(s * 3200, 3200)])


@functools.lru_cache(maxsize=None)
def _deg_kernel():
    return pl.kernel(
        _deg_body,
        out_type=jax.ShapeDtypeStruct((2, 1, _DEG_PAD), jnp.float32),
        mesh=_sc_mesh(),
        scratch_types=[
            pltpu.VMEM((3200,), jnp.float32),
            pltpu.VMEM((_DEG_RPT, _SUB), jnp.int32),
            pltpu.VMEM((_SUB,), jnp.float32),
            pltpu.VMEM_SHARED((_DEG_PAD,), jnp.float32),
        ],
        compiler_params=pltpu.CompilerParams(use_tc_tiling_on_sc=False),
    )


def _scat_body(p_hbm, src_hbm, dst_hbm, out_hbm,
               srcbuf, dstbuf, rows, agg_sp, sem):
    c = lax.axis_index("c")
    s = lax.axis_index("s")
    base = c * _HN

    def _zero(i, _):
        rows[i // 4, pl.ds((i % 4) * 16, 16)] = jnp.zeros((16,), jnp.float32)
        return _
    lax.fori_loop(0, 512, _zero, None)

    def _zs(i, _):
        pltpu.sync_copy(rows, agg_sp.at[pl.ds(s * 1600 + i * 128, 128)])
        return _
    lax.fori_loop(0, 12, _zs, None)
    pltpu.sync_copy(rows.at[pl.ds(0, 64)],
                    agg_sp.at[pl.ds(s * 1600 + 1536, 64)])
    plsc.subcore_barrier()

    def _outer(o, _):
        rb = s * _RPT + o * _STAGE
        pltpu.sync_copy(src_hbm.at[pl.ds(rb, _STAGE)], srcbuf)
        pltpu.sync_copy(dst_hbm.at[pl.ds(rb, _STAGE)], dstbuf)

        def _inner(i, _):
            def _fix(j, _):
                dv = dstbuf[i, pl.ds(j * 16, 16)]
                m = (dv >= base) & (dv < base + _HN)
                loc = jnp.where(m, dv - base, _DUMP + (dv & 255))
                dstbuf[i, pl.ds(j * 16, 16)] = loc
                return _
            lax.fori_loop(0, _SUB // 16, _fix, None)
            pltpu.async_copy(p_hbm.at[srcbuf.at[i]], rows, sem).wait()
            pltpu.sync_copy(rows, agg_sp.at[dstbuf.at[i]], add=True)
            return _
        lax.fori_loop(0, _STAGE, _inner, None)
        return _
    lax.fori_loop(0, _RPT // _STAGE, _outer, None)
    plsc.subcore_barrier()

    pltpu.sync_copy(agg_sp.at[pl.ds(s * 1560, 1560)],
                    out_hbm.at[pl.ds(base + s * 1560, 1560)])

    @pl.when(s < 5)
    def _tail():
        pltpu.sync_copy(agg_sp.at[pl.ds(24960 + s * 8, 8)],
                        out_hbm.at[pl.ds(base + 24960 + s * 8, 8)])


@functools.lru_cache(maxsize=None)
def _scat_kernel():
    return pl.kernel(
        _scat_body,
        out_type=jax.ShapeDtypeStruct((_N, _GH), jnp.float32),
        mesh=_sc_mesh(),
        scratch_types=[
            pltpu.VMEM((_STAGE, _SUB), jnp.int32),
            pltpu.VMEM((_STAGE, _SUB), jnp.int32),
            pltpu.VMEM((_SUB, _GH), jnp.float32),
            pltpu.VMEM_SHARED((_SPROWS, _GH), jnp.float32),
            pltpu.SemaphoreType.DMA,
        ],
        compiler_params=pltpu.CompilerParams(use_tc_tiling_on_sc=False),
    )


def _mm1_body(x_ref, d0_ref, d1_ref, w_ref, p_ref, dinv_ref):
    deg = d0_ref[...] + d1_ref[...] + 1.0
    dinv = lax.rsqrt(deg)
    h = jnp.dot(x_ref[...], w_ref[...], preferred_element_type=jnp.float32)
    p_ref[...] = h * dinv
    dinv_ref[...] = dinv


def _mm1_call(x, d0, d1, w1):
    return pl.pallas_call(
        _mm1_body,
        grid=(_GRID,),
        in_specs=[
            pl.BlockSpec((_BN, _NT), lambda i: (i, 0)),
            pl.BlockSpec((_BN, 1), lambda i: (i, 0)),
            pl.BlockSpec((_BN, 1), lambda i: (i, 0)),
            pl.BlockSpec((_NT, _GH), lambda i: (0, 0)),
        ],
        out_specs=[
            pl.BlockSpec((_BN, _GH), lambda i: (i, 0)),
            pl.BlockSpec((_BN, 1), lambda i: (i, 0)),
        ],
        out_shape=[
            jax.ShapeDtypeStruct((_N, _GH), jnp.float32),
            jax.ShapeDtypeStruct((_N, 1), jnp.float32),
        ],
    )(x, d0, d1, w1)


def _mm2_body(agg_ref, p_ref, dinv_ref, b1_ref, w2_ref, p2_ref):
    dinv = dinv_ref[...]
    out1 = jnp.maximum(
        dinv * (agg_ref[...] + p_ref[...]) + b1_ref[...], 0.0)
    p2_ref[...] = jnp.dot(
        out1, w2_ref[...], preferred_element_type=jnp.float32) * dinv


def _mm2_call(agg1, p1, dinv, b1, w2):
    return pl.pallas_call(
        _mm2_body,
        grid=(_GRID,),
        in_specs=[
            pl.BlockSpec((_BN, _GH), lambda i: (i, 0)),
            pl.BlockSpec((_BN, _GH), lambda i: (i, 0)),
            pl.BlockSpec((_BN, 1), lambda i: (i, 0)),
            pl.BlockSpec((1, _GH), lambda i: (0, 0)),
            pl.BlockSpec((_GH, _GH), lambda i: (0, 0)),
        ],
        out_specs=pl.BlockSpec((_BN, _GH), lambda i: (i, 0)),
        out_shape=jax.ShapeDtypeStruct((_N, _GH), jnp.float32),
    )(agg1, p1, dinv, b1, w2)


def _pool_body(agg_ref, p_ref, dinv_ref, b2_ref, lnw_ref, lnb_ref, batch_ref,
               sum_ref, cnt_ref):
    i = pl.program_id(0)
    h = dinv_ref[...] * (agg_ref[...] + p_ref[...]) + b2_ref[...]
    mu = jnp.mean(h, axis=-1, keepdims=True)
    var = jnp.mean((h - mu) ** 2, axis=-1, keepdims=True)
    hn = (h - mu) / jnp.sqrt(var + 1e-5) * lnw_ref[...] + lnb_ref[...]
    oh = (lax.broadcasted_iota(jnp.int32, (_B, _BN), 0)
          == batch_ref[...].reshape(1, _BN)).astype(jnp.float32)
    ps = lax.dot_general(oh, hn, (((1,), (0,)), ((), ())),
                         preferred_element_type=jnp.float32)
    pc = jnp.sum(oh, axis=1, keepdims=True)

    @pl.when(i == 0)
    def _():
        sum_ref[...] = jnp.zeros_like(sum_ref)
        cnt_ref[...] = jnp.zeros_like(cnt_ref)

    sum_ref[...] += ps
    cnt_ref[...] += pc


def _pool_call(agg2, p2, dinv, b2, lnw, lnb, batch2d):
    return pl.pallas_call(
        _pool_body,
        grid=(_GRID,),
        in_specs=[
            pl.BlockSpec((_BN, _GH), lambda i: (i, 0)),
            pl.BlockSpec((_BN, _GH), lambda i: (i, 0)),
            pl.BlockSpec((_BN, 1), lambda i: (i, 0)),
            pl.BlockSpec((1, _GH), lambda i: (0, 0)),
            pl.BlockSpec((1, _GH), lambda i: (0, 0)),
            pl.BlockSpec((1, _GH), lambda i: (0, 0)),
            pl.BlockSpec((1, 1, _BN), lambda i: (i, 0, 0)),
        ],
        out_specs=[
            pl.BlockSpec((_B, _GH), lambda i: (0, 0)),
            pl.BlockSpec((_B, 1), lambda i: (0, 0)),
        ],
        out_shape=[
            jax.ShapeDtypeStruct((_B, _GH), jnp.float32),
            jax.ShapeDtypeStruct((_B, 1), jnp.float32),
        ],
    )(agg2, p2, dinv, b2, lnw, lnb, batch2d)


def _head_body(text_ref, sum_ref, cnt_ref, wi0_ref, bi0_ref, bh0_ref,
               wi1_ref, bi1_ref, bh1_ref, wnt_ref, bnt_ref, wcp_ref, bcp_ref,
               h1_ref, h2_ref, lg_ref, pr_ref):
    cnt = jnp.maximum(cnt_ref[...], 1.0)
    mean = sum_ref[...] / cnt
    comb = jnp.concatenate([text_ref[...], mean], axis=1)

    def _cell(xv, wih, bih, bhh):
        gi = lax.dot_general(xv, wih, (((1,), (1,)), ((), ())),
                             preferred_element_type=jnp.float32) + bih
        r = jax.nn.sigmoid(gi[:, :_SH] + bhh[:, :_SH])
        z = jax.nn.sigmoid(gi[:, _SH:2 * _SH] + bhh[:, _SH:2 * _SH])
        cg = jnp.tanh(gi[:, 2 * _SH:] + r * bhh[:, 2 * _SH:])
        return (1.0 - z) * cg

    h1 = _cell(comb, wi0_ref[...], bi0_ref[...], bh0_ref[...])
    h2 = _cell(h1, wi1_ref[...], bi1_ref[...], bh1_ref[...])
    h1_ref[...] = h1
    h2_ref[...] = h2
    lg_ref[...] = lax.dot_general(
        h2, wnt_ref[...], (((1,), (1,)), ((), ())),
        preferred_element_type=jnp.float32) + bnt_ref[...]
    pr_ref[...] = jax.nn.sigmoid(lax.dot_general(
        h2, wcp_ref[...], (((1,), (1,)), ((), ())),
        preferred_element_type=jnp.float32) + bcp_ref[...])


def _head_call(text, sums, cnts, wi0, bi0, bh0, wi1, bi1, bh1,
               wnt, bnt, wcp, bcp):
    return pl.pallas_call(
        _head_body,
        out_shape=[
            jax.ShapeDtypeStruct((_B, _SH), jnp.float32),
            jax.ShapeDtypeStruct((_B, _SH), jnp.float32),
            jax.ShapeDtypeStruct((_B, _NT), jnp.float32),
            jax.ShapeDtypeStruct((_B, _MN), jnp.float32),
        ],
    )(text, sums, cnts, wi0, bi0, bh0, wi1, bi1, bh1, wnt, bnt, wcp, bcp)


def _deg_call(dst_deg2d):
    return _deg_kernel()(dst_deg2d)


def _scat_call(p, src2d, dst2d):
    return _scat_kernel()(p, src2d, dst2d)


def kernel(text_embedding, x, edge_index, batch, gcn1_W, gcn1_b, gcn2_W,
           gcn2_b, ln_w, ln_b, W_ih0, W_hh0, b_ih0, b_hh0, W_ih1, W_hh1,
           b_ih1, b_hh1, W_nt, b_nt, W_cp, b_cp):
    src = edge_index[0]
    dst = edge_index[1]
    npad = _EP - _E
    src2d = jnp.concatenate(
        [src, jnp.zeros((npad,), src.dtype)]).reshape(_ROWS, _SUB)
    dst2d = jnp.concatenate(
        [dst, jnp.full((npad,), _N, dst.dtype)]).reshape(_ROWS, _SUB)

    degp = _deg_call(dst2d)
    d0 = degp[0, 0, :_N].reshape(_N, 1)
    d1 = degp[1, 0, :_N].reshape(_N, 1)

    p1, dinv = _mm1_call(x, d0, d1, gcn1_W)
    agg1 = _scat_call(p1, src2d, dst2d)
    p2 = _mm2_call(agg1, p1, dinv, gcn1_b.reshape(1, _GH), gcn2_W)
    agg2 = _scat_call(p2, src2d, dst2d)
    sums, cnts = _pool_call(agg2, p2, dinv, gcn2_b.reshape(1, _GH),
                            ln_w.reshape(1, _GH), ln_b.reshape(1, _GH),
                            batch.reshape(_GRID, 1, _BN))
    h1, h2, logits, probs = _head_call(
        text_embedding, sums, cnts,
        W_ih0, b_ih0.reshape(1, -1), b_hh0.reshape(1, -1),
        W_ih1, b_ih1.reshape(1, -1), b_hh1.reshape(1, -1),
        W_nt, b_nt.reshape(1, -1), W_cp, b_cp.reshape(1, -1))
    return logits, probs, jnp.stack([h1, h2], axis=0)


# speedup vs baseline: 26.0763x; 2.6920x over previous
import functools

import jax
import jax.numpy as jnp
from jax import lax
from jax.experimental import pallas as pl
from jax.experimental.pallas import tpu as pltpu
from jax.experimental.pallas import tpu_sc as plsc

_N = 50000
_E = 800000
_B = 256
_NT = 74
_GH = 64
_SH = 128
_MN = 100

_SUB = 128
_EP = 819200
_ROWS = _EP // _SUB
_RPT = _ROWS // 16
_STAGE = 16
_CB = 17
_DEG_RPT = _ROWS // 32
_DEG_PAD = 51200
_HN = _N // 2
_SPROWS = 25600
_DUMP = 25088

_BN = 2000
_GRID = _N // _BN

@functools.lru_cache(maxsize=None)
def _sc_mesh():
    return plsc.VectorSubcoreMesh(core_axis_name="c", subcore_axis_name="s",
                                  num_cores=2, num_subcores=16)


def _deg_body(dst_hbm, out_hbm, zbuf, idxbuf, onesbuf, deg_sp):
    c = lax.axis_index("c")
    s = lax.axis_index("s")

    def _zero(i, _):
        zbuf[pl.ds(i * 16, 16)] = jnp.zeros((16,), jnp.float32)
        return _
    lax.fori_loop(0, 3200 // 16, _zero, None)
    pltpu.sync_copy(zbuf, deg_sp.at[pl.ds(s * 3200, 3200)])

    def _ones(i, _):
        onesbuf[pl.ds(i * 16, 16)] = jnp.ones((16,), jnp.float32)
        return _
    lax.fori_loop(0, 8, _ones, None)
    plsc.subcore_barrier()

    w = s * 2 + c
    pltpu.sync_copy(dst_hbm.at[pl.ds(w * _DEG_RPT, _DEG_RPT)], idxbuf)

    def _scat(i, _):
        pltpu.sync_copy(onesbuf, deg_sp.at[idxbuf.at[i]], add=True)
        return _
    lax.fori_loop(0, _DEG_RPT, _scat, None)
    plsc.subcore_barrier()
    pltpu.sync_copy(deg_sp.at[pl.ds(s * 3200, 3200)],
                    out_hbm.at[c, 0, pl.ds(s * 3200, 3200)])


@functools.lru_cache(maxsize=None)
def _deg_kernel():
    return pl.kernel(
        _deg_body,
        out_type=jax.ShapeDtypeStruct((2, 1, _DEG_PAD), jnp.float32),
        mesh=_sc_mesh(),
        scratch_types=[
            pltpu.VMEM((3200,), jnp.float32),
            pltpu.VMEM((_DEG_RPT, _SUB), jnp.int32),
            pltpu.VMEM((_SUB,), jnp.float32),
            pltpu.VMEM_SHARED((_DEG_PAD,), jnp.float32),
        ],
        compiler_params=pltpu.CompilerParams(use_tc_tiling_on_sc=False, needs_layout_passes=False),
    )


def _scat_body(p_hbm, src_hbm, dst_hbm, out_hbm,
               srcbuf, dstbuf, csrc, cdst, rows_a, rows_b, agg_sp,
               sem_a, sem_b):
    c = lax.axis_index("c")
    s = lax.axis_index("s")
    base = c * _HN

    def _zero(i, _):
        rows_a[i // 4, pl.ds((i % 4) * 16, 16)] = jnp.zeros((16,), jnp.float32)
        return _
    lax.fori_loop(0, 512, _zero, None)

    def _zs(i, _):
        pltpu.sync_copy(rows_a, agg_sp.at[pl.ds(s * 1600 + i * 128, 128)])
        return _
    lax.fori_loop(0, 12, _zs, None)
    pltpu.sync_copy(rows_a.at[pl.ds(0, 64)],
                    agg_sp.at[pl.ds(s * 1600 + 1536, 64)])
    plsc.subcore_barrier()

    def _flush_pairs(nw):
        def _pair(k, _):
            j0 = 2 * k
            da = pltpu.async_copy(p_hbm.at[csrc.at[j0]], rows_a, sem_a)
            db = pltpu.async_copy(p_hbm.at[csrc.at[j0 + 1]], rows_b, sem_b)
            da.wait()
            pltpu.sync_copy(rows_a, agg_sp.at[cdst.at[j0]], add=True)
            db.wait()
            pltpu.sync_copy(rows_b, agg_sp.at[cdst.at[j0 + 1]], add=True)
            return _
        lax.fori_loop(0, nw >> 1, _pair, None)

        @pl.when((nw & 1) == 1)
        def _odd():
            j = nw - 1
            pltpu.async_copy(p_hbm.at[csrc.at[j]], rows_a, sem_a).wait()
            pltpu.sync_copy(rows_a, agg_sp.at[cdst.at[j]], add=True)

    def _outer(o, off):
        rb = s * _RPT + o * _STAGE
        pltpu.sync_copy(src_hbm.at[pl.ds(rb, _STAGE)], srcbuf)
        pltpu.sync_copy(dst_hbm.at[pl.ds(rb, _STAGE)], dstbuf)

        def _inner(i, off):
            def _app(j, off):
                sv = srcbuf[i, pl.ds(j * 16, 16)]
                dv = dstbuf[i, pl.ds(j * 16, 16)]
                m = (dv >= base) & (dv < base + _HN)
                mi = m.astype(jnp.int32)
                pos = off + plsc.cumsum(mi) - mi
                row = pos >> 7
                col = pos & 127
                plsc.store_scatter(csrc, [row, col], sv, mask=m)
                plsc.store_scatter(cdst, [row, col], dv - base, mask=m)
                return off + jnp.sum(mi)
            return lax.fori_loop(0, _SUB // 16, _app, off)
        off = lax.fori_loop(0, _STAGE, _inner, off)

        nw = off >> 7
        _flush_pairs(nw)

        def _mv(j, _):
            csrc[0, pl.ds(j * 16, 16)] = csrc[nw, pl.ds(j * 16, 16)]
            cdst[0, pl.ds(j * 16, 16)] = cdst[nw, pl.ds(j * 16, 16)]
            return _
        lax.fori_loop(0, 8, _mv, None)
        return off & 127

    off = lax.fori_loop(0, _RPT // _STAGE, _outer, jnp.int32(0))

    @pl.when(off > 0)
    def _tail_win():
        zrow = jnp.zeros((16,), jnp.int32)
        def _pad(j, _):
            lane = j * 16 + lax.iota(jnp.int32, 16)
            mp = lane >= off
            plsc.store_scatter(csrc, [zrow, lane], zrow, mask=mp)
            plsc.store_scatter(cdst, [zrow, lane], _DUMP + lane, mask=mp)
            return _
        lax.fori_loop(0, 8, _pad, None)
        pltpu.async_copy(p_hbm.at[csrc.at[0]], rows_a, sem_a).wait()
        pltpu.sync_copy(rows_a, agg_sp.at[cdst.at[0]], add=True)

    plsc.subcore_barrier()

    pltpu.sync_copy(agg_sp.at[pl.ds(s * 1560, 1560)],
                    out_hbm.at[pl.ds(base + s * 1560, 1560)])

    @pl.when(s < 5)
    def _tail():
        pltpu.sync_copy(agg_sp.at[pl.ds(24960 + s * 8, 8)],
                        out_hbm.at[pl.ds(base + 24960 + s * 8, 8)])


@functools.lru_cache(maxsize=None)
def _scat_kernel():
    return pl.kernel(
        _scat_body,
        out_type=jax.ShapeDtypeStruct((_N, _GH), jnp.float32),
        mesh=_sc_mesh(),
        scratch_types=[
            pltpu.VMEM((_STAGE, _SUB), jnp.int32),
            pltpu.VMEM((_STAGE, _SUB), jnp.int32),
            pltpu.VMEM((_CB, _SUB), jnp.int32),
            pltpu.VMEM((_CB, _SUB), jnp.int32),
            pltpu.VMEM((_SUB, _GH), jnp.float32),
            pltpu.VMEM((_SUB, _GH), jnp.float32),
            pltpu.VMEM_SHARED((_SPROWS, _GH), jnp.float32),
            pltpu.SemaphoreType.DMA,
            pltpu.SemaphoreType.DMA,
        ],
        compiler_params=pltpu.CompilerParams(use_tc_tiling_on_sc=False, needs_layout_passes=False),
    )


def _mm1_body(x_ref, d0_ref, d1_ref, w_ref, p_ref, dinv_ref):
    deg = d0_ref[...] + d1_ref[...] + 1.0
    dinv = lax.rsqrt(deg)
    h = jnp.dot(x_ref[...], w_ref[...], preferred_element_type=jnp.float32)
    p_ref[...] = h * dinv
    dinv_ref[...] = dinv


def _mm1_call(x, d0, d1, w1):
    return pl.pallas_call(
        _mm1_body,
        grid=(_GRID,),
        in_specs=[
            pl.BlockSpec((_BN, _NT), lambda i: (i, 0)),
            pl.BlockSpec((_BN, 1), lambda i: (i, 0)),
            pl.BlockSpec((_BN, 1), lambda i: (i, 0)),
            pl.BlockSpec((_NT, _GH), lambda i: (0, 0)),
        ],
        out_specs=[
            pl.BlockSpec((_BN, _GH), lambda i: (i, 0)),
            pl.BlockSpec((_BN, 1), lambda i: (i, 0)),
        ],
        out_shape=[
            jax.ShapeDtypeStruct((_N, _GH), jnp.float32),
            jax.ShapeDtypeStruct((_N, 1), jnp.float32),
        ],
    )(x, d0, d1, w1)


def _mm2_body(agg_ref, p_ref, dinv_ref, b1_ref, w2_ref, p2_ref):
    dinv = dinv_ref[...]
    out1 = jnp.maximum(
        dinv * (agg_ref[...] + p_ref[...]) + b1_ref[...], 0.0)
    p2_ref[...] = jnp.dot(
        out1, w2_ref[...], preferred_element_type=jnp.float32) * dinv


def _mm2_call(agg1, p1, dinv, b1, w2):
    return pl.pallas_call(
        _mm2_body,
        grid=(_GRID,),
        in_specs=[
            pl.BlockSpec((_BN, _GH), lambda i: (i, 0)),
            pl.BlockSpec((_BN, _GH), lambda i: (i, 0)),
            pl.BlockSpec((_BN, 1), lambda i: (i, 0)),
            pl.BlockSpec((1, _GH), lambda i: (0, 0)),
            pl.BlockSpec((_GH, _GH), lambda i: (0, 0)),
        ],
        out_specs=pl.BlockSpec((_BN, _GH), lambda i: (i, 0)),
        out_shape=jax.ShapeDtypeStruct((_N, _GH), jnp.float32),
    )(agg1, p1, dinv, b1, w2)


def _pool_body(agg_ref, p_ref, dinv_ref, b2_ref, lnw_ref, lnb_ref, batch_ref,
               sum_ref, cnt_ref):
    i = pl.program_id(0)
    h = dinv_ref[...] * (agg_ref[...] + p_ref[...]) + b2_ref[...]
    mu = jnp.mean(h, axis=-1, keepdims=True)
    var = jnp.mean((h - mu) ** 2, axis=-1, keepdims=True)
    hn = (h - mu) / jnp.sqrt(var + 1e-5) * lnw_ref[...] + lnb_ref[...]
    oh = (lax.broadcasted_iota(jnp.int32, (_B, _BN), 0)
          == batch_ref[...].reshape(1, _BN)).astype(jnp.float32)
    ps = lax.dot_general(oh, hn, (((1,), (0,)), ((), ())),
                         preferred_element_type=jnp.float32)
    pc = jnp.sum(oh, axis=1, keepdims=True)

    @pl.when(i == 0)
    def _():
        sum_ref[...] = jnp.zeros_like(sum_ref)
        cnt_ref[...] = jnp.zeros_like(cnt_ref)

    sum_ref[...] += ps
    cnt_ref[...] += pc


def _pool_call(agg2, p2, dinv, b2, lnw, lnb, batch2d):
    return pl.pallas_call(
        _pool_body,
        grid=(_GRID,),
        in_specs=[
            pl.BlockSpec((_BN, _GH), lambda i: (i, 0)),
            pl.BlockSpec((_BN, _GH), lambda i: (i, 0)),
            pl.BlockSpec((_BN, 1), lambda i: (i, 0)),
            pl.BlockSpec((1, _GH), lambda i: (0, 0)),
            pl.BlockSpec((1, _GH), lambda i: (0, 0)),
            pl.BlockSpec((1, _GH), lambda i: (0, 0)),
            pl.BlockSpec((1, 1, _BN), lambda i: (i, 0, 0)),
        ],
        out_specs=[
            pl.BlockSpec((_B, _GH), lambda i: (0, 0)),
            pl.BlockSpec((_B, 1), lambda i: (0, 0)),
        ],
        out_shape=[
            jax.ShapeDtypeStruct((_B, _GH), jnp.float32),
            jax.ShapeDtypeStruct((_B, 1), jnp.float32),
        ],
    )(agg2, p2, dinv, b2, lnw, lnb, batch2d)


def _head_body(text_ref, sum_ref, cnt_ref, wi0_ref, bi0_ref, bh0_ref,
               wi1_ref, bi1_ref, bh1_ref, wnt_ref, bnt_ref, wcp_ref, bcp_ref,
               h1_ref, h2_ref, lg_ref, pr_ref):
    cnt = jnp.maximum(cnt_ref[...], 1.0)
    mean = sum_ref[...] / cnt
    comb = jnp.concatenate([text_ref[...], mean], axis=1)

    def _cell(xv, wih, bih, bhh):
        gi = lax.dot_general(xv, wih, (((1,), (1,)), ((), ())),
                             preferred_element_type=jnp.float32) + bih
        r = jax.nn.sigmoid(gi[:, :_SH] + bhh[:, :_SH])
        z = jax.nn.sigmoid(gi[:, _SH:2 * _SH] + bhh[:, _SH:2 * _SH])
        cg = jnp.tanh(gi[:, 2 * _SH:] + r * bhh[:, 2 * _SH:])
        return (1.0 - z) * cg

    h1 = _cell(comb, wi0_ref[...], bi0_ref[...], bh0_ref[...])
    h2 = _cell(h1, wi1_ref[...], bi1_ref[...], bh1_ref[...])
    h1_ref[...] = h1
    h2_ref[...] = h2
    lg_ref[...] = lax.dot_general(
        h2, wnt_ref[...], (((1,), (1,)), ((), ())),
        preferred_element_type=jnp.float32) + bnt_ref[...]
    pr_ref[...] = jax.nn.sigmoid(lax.dot_general(
        h2, wcp_ref[...], (((1,), (1,)), ((), ())),
        preferred_element_type=jnp.float32) + bcp_ref[...])


def _head_call(text, sums, cnts, wi0, bi0, bh0, wi1, bi1, bh1,
               wnt, bnt, wcp, bcp):
    return pl.pallas_call(
        _head_body,
        out_shape=[
            jax.ShapeDtypeStruct((_B, _SH), jnp.float32),
            jax.ShapeDtypeStruct((_B, _SH), jnp.float32),
            jax.ShapeDtypeStruct((_B, _NT), jnp.float32),
            jax.ShapeDtypeStruct((_B, _MN), jnp.float32),
        ],
    )(text, sums, cnts, wi0, bi0, bh0, wi1, bi1, bh1, wnt, bnt, wcp, bcp)


def _deg_call(dst_deg2d):
    return _deg_kernel()(dst_deg2d)


def _scat_call(p, src2d, dst2d):
    return _scat_kernel()(p, src2d, dst2d)


def kernel(text_embedding, x, edge_index, batch, gcn1_W, gcn1_b, gcn2_W,
           gcn2_b, ln_w, ln_b, W_ih0, W_hh0, b_ih0, b_hh0, W_ih1, W_hh1,
           b_ih1, b_hh1, W_nt, b_nt, W_cp, b_cp):
    src = edge_index[0]
    dst = edge_index[1]
    npad = _EP - _E
    src2d = jnp.concatenate(
        [src, jnp.zeros((npad,), src.dtype)]).reshape(_ROWS, _SUB)
    dst2d = jnp.concatenate(
        [dst, jnp.full((npad,), _N, dst.dtype)]).reshape(_ROWS, _SUB)

    degp = _deg_call(dst2d)
    d0 = degp[0, 0, :_N].reshape(_N, 1)
    d1 = degp[1, 0, :_N].reshape(_N, 1)

    p1, dinv = _mm1_call(x, d0, d1, gcn1_W)
    agg1 = _scat_call(p1, src2d, dst2d)
    p2 = _mm2_call(agg1, p1, dinv, gcn1_b.reshape(1, _GH), gcn2_W)
    agg2 = _scat_call(p2, src2d, dst2d)
    sums, cnts = _pool_call(agg2, p2, dinv, gcn2_b.reshape(1, _GH),
                            ln_w.reshape(1, _GH), ln_b.reshape(1, _GH),
                            batch.reshape(_GRID, 1, _BN))
    h1, h2, logits, probs = _head_call(
        text_embedding, sums, cnts,
        W_ih0, b_ih0.reshape(1, -1), b_hh0.reshape(1, -1),
        W_ih1, b_ih1.reshape(1, -1), b_hh1.reshape(1, -1),
        W_nt, b_nt.reshape(1, -1), W_cp, b_cp.reshape(1, -1))
    return logits, probs, jnp.stack([h1, h2], axis=0)


# speedup vs baseline: 26.4754x; 1.0153x over previous
import functools

import jax
import jax.numpy as jnp
from jax import lax
from jax.experimental import pallas as pl
from jax.experimental.pallas import tpu as pltpu
from jax.experimental.pallas import tpu_sc as plsc

_N = 50000
_E = 800000
_B = 256
_NT = 74
_GH = 64
_SH = 128
_MN = 100

_SUB = 128
_EP = 819200
_ROWS = _EP // _SUB
_RPT = _ROWS // 16
_STAGE = 16
_WIN = 64
_CB = 34
_DEG_RPT = _ROWS // 32
_DEG_PAD = 51200
_HN = _N // 2
_SPROWS = 25216
_DUMP = 25088

_BN = 2000
_GRID = _N // _BN

@functools.lru_cache(maxsize=None)
def _sc_mesh():
    return plsc.VectorSubcoreMesh(core_axis_name="c", subcore_axis_name="s",
                                  num_cores=2, num_subcores=16)


def _deg_body(dst_hbm, out_hbm, zbuf, idxbuf, onesbuf, deg_sp):
    c = lax.axis_index("c")
    s = lax.axis_index("s")

    def _zero(i, _):
        zbuf[pl.ds(i * 16, 16)] = jnp.zeros((16,), jnp.float32)
        return _
    lax.fori_loop(0, 3200 // 16, _zero, None)
    pltpu.sync_copy(zbuf, deg_sp.at[pl.ds(s * 3200, 3200)])

    def _ones(i, _):
        onesbuf[pl.ds(i * 16, 16)] = jnp.ones((16,), jnp.float32)
        return _
    lax.fori_loop(0, 8, _ones, None)
    plsc.subcore_barrier()

    w = s * 2 + c
    pltpu.sync_copy(dst_hbm.at[pl.ds(w * _DEG_RPT, _DEG_RPT)], idxbuf)

    def _scat(i, _):
        pltpu.sync_copy(onesbuf, deg_sp.at[idxbuf.at[i]], add=True)
        return _
    lax.fori_loop(0, _DEG_RPT, _scat, None)
    plsc.subcore_barrier()
    pltpu.sync_copy(deg_sp.at[pl.ds(s * 3200, 3200)],
                    out_hbm.at[c, 0, pl.ds(s * 3200, 3200)])


@functools.lru_cache(maxsize=None)
def _deg_kernel():
    return pl.kernel(
        _deg_body,
        out_type=jax.ShapeDtypeStruct((2, 1, _DEG_PAD), jnp.float32),
        mesh=_sc_mesh(),
        scratch_types=[
            pltpu.VMEM((3200,), jnp.float32),
            pltpu.VMEM((_DEG_RPT, _SUB), jnp.int32),
            pltpu.VMEM((_SUB,), jnp.float32),
            pltpu.VMEM_SHARED((_DEG_PAD,), jnp.float32),
        ],
        compiler_params=pltpu.CompilerParams(use_tc_tiling_on_sc=False, needs_layout_passes=False),
    )


def _scat_body(p_hbm, src_hbm, dst_hbm, out_hbm,
               srcbuf, dstbuf, csrc, cdst, r0, r1, r2, r3, agg_sp,
               m0, m1, m2, m3):
    c = lax.axis_index("c")
    s = lax.axis_index("s")
    base = c * _HN

    def _zero(i, _):
        r0[i // 4, pl.ds((i % 4) * 16, 16)] = jnp.zeros((16,), jnp.float32)
        return _
    lax.fori_loop(0, 256, _zero, None)

    def _zs(i, _):
        pltpu.sync_copy(r0, agg_sp.at[pl.ds(s * 1576 + i * 64, 64)])
        return _
    lax.fori_loop(0, 24, _zs, None)
    pltpu.sync_copy(r0.at[pl.ds(0, 40)],
                    agg_sp.at[pl.ds(s * 1576 + 1536, 40)])
    plsc.subcore_barrier()

    def _one(j, rbuf, sem):
        pltpu.async_copy(p_hbm.at[csrc.at[j]], rbuf, sem).wait()
        pltpu.sync_copy(rbuf, agg_sp.at[cdst.at[j]], add=True)

    def _flush(nw):
        def _quad(q, _):
            j0 = 4 * q
            d0 = pltpu.async_copy(p_hbm.at[csrc.at[j0]], r0, m0)
            d1 = pltpu.async_copy(p_hbm.at[csrc.at[j0 + 1]], r1, m1)
            d2 = pltpu.async_copy(p_hbm.at[csrc.at[j0 + 2]], r2, m2)
            d3 = pltpu.async_copy(p_hbm.at[csrc.at[j0 + 3]], r3, m3)
            d0.wait()
            pltpu.sync_copy(r0, agg_sp.at[cdst.at[j0]], add=True)
            d1.wait()
            pltpu.sync_copy(r1, agg_sp.at[cdst.at[j0 + 1]], add=True)
            d2.wait()
            pltpu.sync_copy(r2, agg_sp.at[cdst.at[j0 + 2]], add=True)
            d3.wait()
            pltpu.sync_copy(r3, agg_sp.at[cdst.at[j0 + 3]], add=True)
            return _
        lax.fori_loop(0, nw >> 2, _quad, None)

        def _rem(j, _):
            _one(j, r0, m0)
            return _
        lax.fori_loop((nw >> 2) * 4, nw, _rem, None)

    def _outer(o, off):
        rb = s * _RPT + o * _STAGE
        pltpu.sync_copy(src_hbm.at[pl.ds(rb, _STAGE)], srcbuf)
        pltpu.sync_copy(dst_hbm.at[pl.ds(rb, _STAGE)], dstbuf)

        def _inner(i, off):
            def _app(j, off):
                sv = srcbuf[i, pl.ds(j * 16, 16)]
                dv = dstbuf[i, pl.ds(j * 16, 16)]
                m = (dv >= base) & (dv < base + _HN)
                mi = m.astype(jnp.int32)
                pos = off + plsc.cumsum(mi) - mi
                row = pos >> 6
                col = pos & 63
                plsc.store_scatter(csrc, [row, col], sv, mask=m)
                plsc.store_scatter(cdst, [row, col], dv - base, mask=m)
                return off + jnp.sum(mi)
            return lax.fori_loop(0, _SUB // 16, _app, off)
        off = lax.fori_loop(0, _STAGE, _inner, off)

        nw = off >> 6
        _flush(nw)

        def _mv(j, _):
            csrc[0, pl.ds(j * 16, 16)] = csrc[nw, pl.ds(j * 16, 16)]
            cdst[0, pl.ds(j * 16, 16)] = cdst[nw, pl.ds(j * 16, 16)]
            return _
        lax.fori_loop(0, 4, _mv, None)
        return off & 63

    off = lax.fori_loop(0, _RPT // _STAGE, _outer, jnp.int32(0))

    @pl.when(off > 0)
    def _tail_win():
        zrow = jnp.zeros((16,), jnp.int32)
        def _pad(j, _):
            lane = j * 16 + lax.iota(jnp.int32, 16)
            mp = lane >= off
            plsc.store_scatter(csrc, [zrow, lane], zrow, mask=mp)
            plsc.store_scatter(cdst, [zrow, lane], _DUMP + lane, mask=mp)
            return _
        lax.fori_loop(0, 4, _pad, None)
        _one(0, r0, m0)

    plsc.subcore_barrier()

    pltpu.sync_copy(agg_sp.at[pl.ds(s * 1560, 1560)],
                    out_hbm.at[pl.ds(base + s * 1560, 1560)])

    @pl.when(s < 5)
    def _tail():
        pltpu.sync_copy(agg_sp.at[pl.ds(24960 + s * 8, 8)],
                        out_hbm.at[pl.ds(base + 24960 + s * 8, 8)])


@functools.lru_cache(maxsize=None)
def _scat_kernel():
    return pl.kernel(
        _scat_body,
        out_type=jax.ShapeDtypeStruct((_N, _GH), jnp.float32),
        mesh=_sc_mesh(),
        scratch_types=[
            pltpu.VMEM((_STAGE, _SUB), jnp.int32),
            pltpu.VMEM((_STAGE, _SUB), jnp.int32),
            pltpu.VMEM((_CB, _WIN), jnp.int32),
            pltpu.VMEM((_CB, _WIN), jnp.int32),
            pltpu.VMEM((_WIN, _GH), jnp.float32),
            pltpu.VMEM((_WIN, _GH), jnp.float32),
            pltpu.VMEM((_WIN, _GH), jnp.float32),
            pltpu.VMEM((_WIN, _GH), jnp.float32),
            pltpu.VMEM_SHARED((_SPROWS, _GH), jnp.float32),
            pltpu.SemaphoreType.DMA,
            pltpu.SemaphoreType.DMA,
            pltpu.SemaphoreType.DMA,
            pltpu.SemaphoreType.DMA,
        ],
        compiler_params=pltpu.CompilerParams(use_tc_tiling_on_sc=False, needs_layout_passes=False),
    )


def _mm1_body(x_ref, d0_ref, d1_ref, w_ref, p_ref, dinv_ref):
    deg = d0_ref[...] + d1_ref[...] + 1.0
    dinv = lax.rsqrt(deg)
    h = jnp.dot(x_ref[...], w_ref[...], preferred_element_type=jnp.float32)
    p_ref[...] = h * dinv
    dinv_ref[...] = dinv


def _mm1_call(x, d0, d1, w1):
    return pl.pallas_call(
        _mm1_body,
        grid=(_GRID,),
        in_specs=[
            pl.BlockSpec((_BN, _NT), lambda i: (i, 0)),
            pl.BlockSpec((_BN, 1), lambda i: (i, 0)),
            pl.BlockSpec((_BN, 1), lambda i: (i, 0)),
            pl.BlockSpec((_NT, _GH), lambda i: (0, 0)),
        ],
        out_specs=[
            pl.BlockSpec((_BN, _GH), lambda i: (i, 0)),
            pl.BlockSpec((_BN, 1), lambda i: (i, 0)),
        ],
        out_shape=[
            jax.ShapeDtypeStruct((_N, _GH), jnp.float32),
            jax.ShapeDtypeStruct((_N, 1), jnp.float32),
        ],
    )(x, d0, d1, w1)


def _mm2_body(agg_ref, p_ref, dinv_ref, b1_ref, w2_ref, p2_ref):
    dinv = dinv_ref[...]
    out1 = jnp.maximum(
        dinv * (agg_ref[...] + p_ref[...]) + b1_ref[...], 0.0)
    p2_ref[...] = jnp.dot(
        out1, w2_ref[...], preferred_element_type=jnp.float32) * dinv


def _mm2_call(agg1, p1, dinv, b1, w2):
    return pl.pallas_call(
        _mm2_body,
        grid=(_GRID,),
        in_specs=[
            pl.BlockSpec((_BN, _GH), lambda i: (i, 0)),
            pl.BlockSpec((_BN, _GH), lambda i: (i, 0)),
            pl.BlockSpec((_BN, 1), lambda i: (i, 0)),
            pl.BlockSpec((1, _GH), lambda i: (0, 0)),
            pl.BlockSpec((_GH, _GH), lambda i: (0, 0)),
        ],
        out_specs=pl.BlockSpec((_BN, _GH), lambda i: (i, 0)),
        out_shape=jax.ShapeDtypeStruct((_N, _GH), jnp.float32),
    )(agg1, p1, dinv, b1, w2)


def _pool_body(agg_ref, p_ref, dinv_ref, b2_ref, lnw_ref, lnb_ref, batch_ref,
               sum_ref, cnt_ref):
    i = pl.program_id(0)
    h = dinv_ref[...] * (agg_ref[...] + p_ref[...]) + b2_ref[...]
    mu = jnp.mean(h, axis=-1, keepdims=True)
    var = jnp.mean((h - mu) ** 2, axis=-1, keepdims=True)
    hn = (h - mu) / jnp.sqrt(var + 1e-5) * lnw_ref[...] + lnb_ref[...]
    oh = (lax.broadcasted_iota(jnp.int32, (_B, _BN), 0)
          == batch_ref[...].reshape(1, _BN)).astype(jnp.float32)
    ps = lax.dot_general(oh, hn, (((1,), (0,)), ((), ())),
                         preferred_element_type=jnp.float32)
    pc = jnp.sum(oh, axis=1, keepdims=True)

    @pl.when(i == 0)
    def _():
        sum_ref[...] = jnp.zeros_like(sum_ref)
        cnt_ref[...] = jnp.zeros_like(cnt_ref)

    sum_ref[...] += ps
    cnt_ref[...] += pc


def _pool_call(agg2, p2, dinv, b2, lnw, lnb, batch2d):
    return pl.pallas_call(
        _pool_body,
        grid=(_GRID,),
        in_specs=[
            pl.BlockSpec((_BN, _GH), lambda i: (i, 0)),
            pl.BlockSpec((_BN, _GH), lambda i: (i, 0)),
            pl.BlockSpec((_BN, 1), lambda i: (i, 0)),
            pl.BlockSpec((1, _GH), lambda i: (0, 0)),
            pl.BlockSpec((1, _GH), lambda i: (0, 0)),
            pl.BlockSpec((1, _GH), lambda i: (0, 0)),
            pl.BlockSpec((1, 1, _BN), lambda i: (i, 0, 0)),
        ],
        out_specs=[
            pl.BlockSpec((_B, _GH), lambda i: (0, 0)),
            pl.BlockSpec((_B, 1), lambda i: (0, 0)),
        ],
        out_shape=[
            jax.ShapeDtypeStruct((_B, _GH), jnp.float32),
            jax.ShapeDtypeStruct((_B, 1), jnp.float32),
        ],
    )(agg2, p2, dinv, b2, lnw, lnb, batch2d)


def _head_body(text_ref, sum_ref, cnt_ref, wi0_ref, bi0_ref, bh0_ref,
               wi1_ref, bi1_ref, bh1_ref, wnt_ref, bnt_ref, wcp_ref, bcp_ref,
               h1_ref, h2_ref, lg_ref, pr_ref):
    cnt = jnp.maximum(cnt_ref[...], 1.0)
    mean = sum_ref[...] / cnt
    comb = jnp.concatenate([text_ref[...], mean], axis=1)

    def _cell(xv, wih, bih, bhh):
        gi = lax.dot_general(xv, wih, (((1,), (1,)), ((), ())),
                             preferred_element_type=jnp.float32) + bih
        r = jax.nn.sigmoid(gi[:, :_SH] + bhh[:, :_SH])
        z = jax.nn.sigmoid(gi[:, _SH:2 * _SH] + bhh[:, _SH:2 * _SH])
        cg = jnp.tanh(gi[:, 2 * _SH:] + r * bhh[:, 2 * _SH:])
        return (1.0 - z) * cg

    h1 = _cell(comb, wi0_ref[...], bi0_ref[...], bh0_ref[...])
    h2 = _cell(h1, wi1_ref[...], bi1_ref[...], bh1_ref[...])
    h1_ref[...] = h1
    h2_ref[...] = h2
    lg_ref[...] = lax.dot_general(
        h2, wnt_ref[...], (((1,), (1,)), ((), ())),
        preferred_element_type=jnp.float32) + bnt_ref[...]
    pr_ref[...] = jax.nn.sigmoid(lax.dot_general(
        h2, wcp_ref[...], (((1,), (1,)), ((), ())),
        preferred_element_type=jnp.float32) + bcp_ref[...])


def _head_call(text, sums, cnts, wi0, bi0, bh0, wi1, bi1, bh1,
               wnt, bnt, wcp, bcp):
    return pl.pallas_call(
        _head_body,
        out_shape=[
            jax.ShapeDtypeStruct((_B, _SH), jnp.float32),
            jax.ShapeDtypeStruct((_B, _SH), jnp.float32),
            jax.ShapeDtypeStruct((_B, _NT), jnp.float32),
            jax.ShapeDtypeStruct((_B, _MN), jnp.float32),
        ],
    )(text, sums, cnts, wi0, bi0, bh0, wi1, bi1, bh1, wnt, bnt, wcp, bcp)


def _deg_call(dst_deg2d):
    return _deg_kernel()(dst_deg2d)


def _scat_call(p, src2d, dst2d):
    return _scat_kernel()(p, src2d, dst2d)


def kernel(text_embedding, x, edge_index, batch, gcn1_W, gcn1_b, gcn2_W,
           gcn2_b, ln_w, ln_b, W_ih0, W_hh0, b_ih0, b_hh0, W_ih1, W_hh1,
           b_ih1, b_hh1, W_nt, b_nt, W_cp, b_cp):
    src = edge_index[0]
    dst = edge_index[1]
    npad = _EP - _E
    src2d = jnp.concatenate(
        [src, jnp.zeros((npad,), src.dtype)]).reshape(_ROWS, _SUB)
    dst2d = jnp.concatenate(
        [dst, jnp.full((npad,), _N, dst.dtype)]).reshape(_ROWS, _SUB)

    degp = _deg_call(dst2d)
    d0 = degp[0, 0, :_N].reshape(_N, 1)
    d1 = degp[1, 0, :_N].reshape(_N, 1)

    p1, dinv = _mm1_call(x, d0, d1, gcn1_W)
    agg1 = _scat_call(p1, src2d, dst2d)
    p2 = _mm2_call(agg1, p1, dinv, gcn1_b.reshape(1, _GH), gcn2_W)
    agg2 = _scat_call(p2, src2d, dst2d)
    sums, cnts = _pool_call(agg2, p2, dinv, gcn2_b.reshape(1, _GH),
                            ln_w.reshape(1, _GH), ln_b.reshape(1, _GH),
                            batch.reshape(_GRID, 1, _BN))
    h1, h2, logits, probs = _head_call(
        text_embedding, sums, cnts,
        W_ih0, b_ih0.reshape(1, -1), b_hh0.reshape(1, -1),
        W_ih1, b_ih1.reshape(1, -1), b_hh1.reshape(1, -1),
        W_nt, b_nt.reshape(1, -1), W_cp, b_cp.reshape(1, -1))
    return logits, probs, jnp.stack([h1, h2], axis=0)


# speedup vs baseline: 26.7616x; 1.0108x over previous
import functools

import jax
import jax.numpy as jnp
from jax import lax
from jax.experimental import pallas as pl
from jax.experimental.pallas import tpu as pltpu
from jax.experimental.pallas import tpu_sc as plsc

_N = 50000
_E = 800000
_B = 256
_NT = 74
_GH = 64
_SH = 128
_MN = 100

_SUB = 128
_EP = 819200
_ROWS = _EP // _SUB
_RPT = _ROWS // 16
_STAGE = 16
_WIN = 64
_CB = 34
_DEG_RPT = _ROWS // 32
_DEG_PAD = 51200
_HN = _N // 2
_SPROWS = 25216
_DUMP = 25088

_BN = 10000
_GRID = _N // _BN

@functools.lru_cache(maxsize=None)
def _sc_mesh():
    return plsc.VectorSubcoreMesh(core_axis_name="c", subcore_axis_name="s",
                                  num_cores=2, num_subcores=16)


def _deg_body(dst_hbm, out_hbm, zbuf, idxbuf, onesbuf, deg_sp):
    c = lax.axis_index("c")
    s = lax.axis_index("s")

    def _zero(i, _):
        zbuf[pl.ds(i * 16, 16)] = jnp.zeros((16,), jnp.float32)
        return _
    lax.fori_loop(0, 3200 // 16, _zero, None)
    pltpu.sync_copy(zbuf, deg_sp.at[pl.ds(s * 3200, 3200)])

    def _ones(i, _):
        onesbuf[pl.ds(i * 16, 16)] = jnp.ones((16,), jnp.float32)
        return _
    lax.fori_loop(0, 8, _ones, None)
    plsc.subcore_barrier()

    w = s * 2 + c
    pltpu.sync_copy(dst_hbm.at[pl.ds(w * _DEG_RPT, _DEG_RPT)], idxbuf)

    def _scat(i, _):
        pltpu.sync_copy(onesbuf, deg_sp.at[idxbuf.at[i]], add=True)
        return _
    lax.fori_loop(0, _DEG_RPT, _scat, None)
    plsc.subcore_barrier()
    pltpu.sync_copy(deg_sp.at[pl.ds(s * 3200, 3200)],
                    out_hbm.at[c, 0, pl.ds(s * 3200, 3200)])


@functools.lru_cache(maxsize=None)
def _deg_kernel():
    return pl.kernel(
        _deg_body,
        out_type=jax.ShapeDtypeStruct((2, 1, _DEG_PAD), jnp.float32),
        mesh=_sc_mesh(),
        scratch_types=[
            pltpu.VMEM((3200,), jnp.float32),
            pltpu.VMEM((_DEG_RPT, _SUB), jnp.int32),
            pltpu.VMEM((_SUB,), jnp.float32),
            pltpu.VMEM_SHARED((_DEG_PAD,), jnp.float32),
        ],
        compiler_params=pltpu.CompilerParams(use_tc_tiling_on_sc=False, needs_layout_passes=False),
    )


def _scat_body(p_hbm, src_hbm, dst_hbm, out_hbm,
               srcbuf, dstbuf, csrc, cdst, r0, r1, r2, r3, agg_sp,
               m0, m1, m2, m3):
    c = lax.axis_index("c")
    s = lax.axis_index("s")
    base = c * _HN

    def _zero(i, _):
        r0[i // 4, pl.ds((i % 4) * 16, 16)] = jnp.zeros((16,), jnp.float32)
        return _
    lax.fori_loop(0, 256, _zero, None)

    def _zs(i, _):
        pltpu.sync_copy(r0, agg_sp.at[pl.ds(s * 1576 + i * 64, 64)])
        return _
    lax.fori_loop(0, 24, _zs, None)
    pltpu.sync_copy(r0.at[pl.ds(0, 40)],
                    agg_sp.at[pl.ds(s * 1576 + 1536, 40)])
    plsc.subcore_barrier()

    def _one(j, rbuf, sem):
        pltpu.async_copy(p_hbm.at[csrc.at[j]], rbuf, sem).wait()
        pltpu.sync_copy(rbuf, agg_sp.at[cdst.at[j]], add=True)

    def _flush(nw):
        def _quad(q, _):
            j0 = 4 * q
            d0 = pltpu.async_copy(p_hbm.at[csrc.at[j0]], r0, m0)
            d1 = pltpu.async_copy(p_hbm.at[csrc.at[j0 + 1]], r1, m1)
            d2 = pltpu.async_copy(p_hbm.at[csrc.at[j0 + 2]], r2, m2)
            d3 = pltpu.async_copy(p_hbm.at[csrc.at[j0 + 3]], r3, m3)
            d0.wait()
            pltpu.sync_copy(r0, agg_sp.at[cdst.at[j0]], add=True)
            d1.wait()
            pltpu.sync_copy(r1, agg_sp.at[cdst.at[j0 + 1]], add=True)
            d2.wait()
            pltpu.sync_copy(r2, agg_sp.at[cdst.at[j0 + 2]], add=True)
            d3.wait()
            pltpu.sync_copy(r3, agg_sp.at[cdst.at[j0 + 3]], add=True)
            return _
        lax.fori_loop(0, nw >> 2, _quad, None)

        def _rem(j, _):
            _one(j, r0, m0)
            return _
        lax.fori_loop((nw >> 2) * 4, nw, _rem, None)

    def _outer(o, off):
        rb = s * _RPT + o * _STAGE
        pltpu.sync_copy(src_hbm.at[pl.ds(rb, _STAGE)], srcbuf)
        pltpu.sync_copy(dst_hbm.at[pl.ds(rb, _STAGE)], dstbuf)

        def _inner(i, off):
            def _app(j, off):
                sv = srcbuf[i, pl.ds(j * 16, 16)]
                dv = dstbuf[i, pl.ds(j * 16, 16)]
                m = (dv >= base) & (dv < base + _HN)
                mi = m.astype(jnp.int32)
                pos = off + plsc.cumsum(mi) - mi
                row = pos >> 6
                col = pos & 63
                plsc.store_scatter(csrc, [row, col], sv, mask=m)
                plsc.store_scatter(cdst, [row, col], dv - base, mask=m)
                return off + jnp.sum(mi)
            return lax.fori_loop(0, _SUB // 16, _app, off)
        off = lax.fori_loop(0, _STAGE, _inner, off)

        nw = off >> 6
        _flush(nw)

        def _mv(j, _):
            csrc[0, pl.ds(j * 16, 16)] = csrc[nw, pl.ds(j * 16, 16)]
            cdst[0, pl.ds(j * 16, 16)] = cdst[nw, pl.ds(j * 16, 16)]
            return _
        lax.fori_loop(0, 4, _mv, None)
        return off & 63

    off = lax.fori_loop(0, _RPT // _STAGE, _outer, jnp.int32(0))

    @pl.when(off > 0)
    def _tail_win():
        zrow = jnp.zeros((16,), jnp.int32)
        def _pad(j, _):
            lane = j * 16 + lax.iota(jnp.int32, 16)
            mp = lane >= off
            plsc.store_scatter(csrc, [zrow, lane], zrow, mask=mp)
            plsc.store_scatter(cdst, [zrow, lane], _DUMP + lane, mask=mp)
            return _
        lax.fori_loop(0, 4, _pad, None)
        _one(0, r0, m0)

    plsc.subcore_barrier()

    pltpu.sync_copy(agg_sp.at[pl.ds(s * 1560, 1560)],
                    out_hbm.at[pl.ds(base + s * 1560, 1560)])

    @pl.when(s < 5)
    def _tail():
        pltpu.sync_copy(agg_sp.at[pl.ds(24960 + s * 8, 8)],
                        out_hbm.at[pl.ds(base + 24960 + s * 8, 8)])


@functools.lru_cache(maxsize=None)
def _scat_kernel():
    return pl.kernel(
        _scat_body,
        out_type=jax.ShapeDtypeStruct((_N, _GH), jnp.float32),
        mesh=_sc_mesh(),
        scratch_types=[
            pltpu.VMEM((_STAGE, _SUB), jnp.int32),
            pltpu.VMEM((_STAGE, _SUB), jnp.int32),
            pltpu.VMEM((_CB, _WIN), jnp.int32),
            pltpu.VMEM((_CB, _WIN), jnp.int32),
            pltpu.VMEM((_WIN, _GH), jnp.float32),
            pltpu.VMEM((_WIN, _GH), jnp.float32),
            pltpu.VMEM((_WIN, _GH), jnp.float32),
            pltpu.VMEM((_WIN, _GH), jnp.float32),
            pltpu.VMEM_SHARED((_SPROWS, _GH), jnp.float32),
            pltpu.SemaphoreType.DMA,
            pltpu.SemaphoreType.DMA,
            pltpu.SemaphoreType.DMA,
            pltpu.SemaphoreType.DMA,
        ],
        compiler_params=pltpu.CompilerParams(use_tc_tiling_on_sc=False, needs_layout_passes=False),
    )


def _mm1_body(x_ref, d0_ref, d1_ref, w_ref, p_ref, dinv_ref):
    deg = d0_ref[...] + d1_ref[...] + 1.0
    dinv = lax.rsqrt(deg)
    h = jnp.dot(x_ref[...], w_ref[...], preferred_element_type=jnp.float32)
    p_ref[...] = h * dinv
    dinv_ref[...] = dinv


def _mm1_call(x, d0, d1, w1):
    return pl.pallas_call(
        _mm1_body,
        grid=(_GRID,),
        in_specs=[
            pl.BlockSpec((_BN, _NT), lambda i: (i, 0)),
            pl.BlockSpec((_BN, 1), lambda i: (i, 0)),
            pl.BlockSpec((_BN, 1), lambda i: (i, 0)),
            pl.BlockSpec((_NT, _GH), lambda i: (0, 0)),
        ],
        out_specs=[
            pl.BlockSpec((_BN, _GH), lambda i: (i, 0)),
            pl.BlockSpec((_BN, 1), lambda i: (i, 0)),
        ],
        out_shape=[
            jax.ShapeDtypeStruct((_N, _GH), jnp.float32),
            jax.ShapeDtypeStruct((_N, 1), jnp.float32),
        ],
    )(x, d0, d1, w1)


def _mm2_body(agg_ref, p_ref, dinv_ref, b1_ref, w2_ref, p2_ref):
    dinv = dinv_ref[...]
    out1 = jnp.maximum(
        dinv * (agg_ref[...] + p_ref[...]) + b1_ref[...], 0.0)
    p2_ref[...] = jnp.dot(
        out1, w2_ref[...], preferred_element_type=jnp.float32) * dinv


def _mm2_call(agg1, p1, dinv, b1, w2):
    return pl.pallas_call(
        _mm2_body,
        grid=(_GRID,),
        in_specs=[
            pl.BlockSpec((_BN, _GH), lambda i: (i, 0)),
            pl.BlockSpec((_BN, _GH), lambda i: (i, 0)),
            pl.BlockSpec((_BN, 1), lambda i: (i, 0)),
            pl.BlockSpec((1, _GH), lambda i: (0, 0)),
            pl.BlockSpec((_GH, _GH), lambda i: (0, 0)),
        ],
        out_specs=pl.BlockSpec((_BN, _GH), lambda i: (i, 0)),
        out_shape=jax.ShapeDtypeStruct((_N, _GH), jnp.float32),
    )(agg1, p1, dinv, b1, w2)


def _pool_body(agg_ref, p_ref, dinv_ref, b2_ref, lnw_ref, lnb_ref, batch_ref,
               sum_ref, cnt_ref):
    i = pl.program_id(0)
    h = dinv_ref[...] * (agg_ref[...] + p_ref[...]) + b2_ref[...]
    mu = jnp.mean(h, axis=-1, keepdims=True)
    var = jnp.mean((h - mu) ** 2, axis=-1, keepdims=True)
    hn = (h - mu) / jnp.sqrt(var + 1e-5) * lnw_ref[...] + lnb_ref[...]
    oh = (lax.broadcasted_iota(jnp.int32, (_B, _BN), 0)
          == batch_ref[...].reshape(1, _BN)).astype(jnp.float32)
    ps = lax.dot_general(oh, hn, (((1,), (0,)), ((), ())),
                         preferred_element_type=jnp.float32)
    pc = jnp.sum(oh, axis=1, keepdims=True)

    @pl.when(i == 0)
    def _():
        sum_ref[...] = jnp.zeros_like(sum_ref)
        cnt_ref[...] = jnp.zeros_like(cnt_ref)

    sum_ref[...] += ps
    cnt_ref[...] += pc


def _pool_call(agg2, p2, dinv, b2, lnw, lnb, batch2d):
    return pl.pallas_call(
        _pool_body,
        grid=(_GRID,),
        in_specs=[
            pl.BlockSpec((_BN, _GH), lambda i: (i, 0)),
            pl.BlockSpec((_BN, _GH), lambda i: (i, 0)),
            pl.BlockSpec((_BN, 1), lambda i: (i, 0)),
            pl.BlockSpec((1, _GH), lambda i: (0, 0)),
            pl.BlockSpec((1, _GH), lambda i: (0, 0)),
            pl.BlockSpec((1, _GH), lambda i: (0, 0)),
            pl.BlockSpec((1, 1, _BN), lambda i: (i, 0, 0)),
        ],
        out_specs=[
            pl.BlockSpec((_B, _GH), lambda i: (0, 0)),
            pl.BlockSpec((_B, 1), lambda i: (0, 0)),
        ],
        out_shape=[
            jax.ShapeDtypeStruct((_B, _GH), jnp.float32),
            jax.ShapeDtypeStruct((_B, 1), jnp.float32),
        ],
    )(agg2, p2, dinv, b2, lnw, lnb, batch2d)


def _head_body(text_ref, sum_ref, cnt_ref, wi0_ref, bi0_ref, bh0_ref,
               wi1_ref, bi1_ref, bh1_ref, wnt_ref, bnt_ref, wcp_ref, bcp_ref,
               h1_ref, h2_ref, lg_ref, pr_ref):
    cnt = jnp.maximum(cnt_ref[...], 1.0)
    mean = sum_ref[...] / cnt
    comb = jnp.concatenate([text_ref[...], mean], axis=1)

    def _cell(xv, wih, bih, bhh):
        gi = lax.dot_general(xv, wih, (((1,), (1,)), ((), ())),
                             preferred_element_type=jnp.float32) + bih
        r = jax.nn.sigmoid(gi[:, :_SH] + bhh[:, :_SH])
        z = jax.nn.sigmoid(gi[:, _SH:2 * _SH] + bhh[:, _SH:2 * _SH])
        cg = jnp.tanh(gi[:, 2 * _SH:] + r * bhh[:, 2 * _SH:])
        return (1.0 - z) * cg

    h1 = _cell(comb, wi0_ref[...], bi0_ref[...], bh0_ref[...])
    h2 = _cell(h1, wi1_ref[...], bi1_ref[...], bh1_ref[...])
    h1_ref[...] = h1
    h2_ref[...] = h2
    lg_ref[...] = lax.dot_general(
        h2, wnt_ref[...], (((1,), (1,)), ((), ())),
        preferred_element_type=jnp.float32) + bnt_ref[...]
    pr_ref[...] = jax.nn.sigmoid(lax.dot_general(
        h2, wcp_ref[...], (((1,), (1,)), ((), ())),
        preferred_element_type=jnp.float32) + bcp_ref[...])


def _head_call(text, sums, cnts, wi0, bi0, bh0, wi1, bi1, bh1,
               wnt, bnt, wcp, bcp):
    return pl.pallas_call(
        _head_body,
        out_shape=[
            jax.ShapeDtypeStruct((_B, _SH), jnp.float32),
            jax.ShapeDtypeStruct((_B, _SH), jnp.float32),
            jax.ShapeDtypeStruct((_B, _NT), jnp.float32),
            jax.ShapeDtypeStruct((_B, _MN), jnp.float32),
        ],
    )(text, sums, cnts, wi0, bi0, bh0, wi1, bi1, bh1, wnt, bnt, wcp, bcp)


def _deg_call(dst_deg2d):
    return _deg_kernel()(dst_deg2d)


def _scat_call(p, src2d, dst2d):
    return _scat_kernel()(p, src2d, dst2d)


def kernel(text_embedding, x, edge_index, batch, gcn1_W, gcn1_b, gcn2_W,
           gcn2_b, ln_w, ln_b, W_ih0, W_hh0, b_ih0, b_hh0, W_ih1, W_hh1,
           b_ih1, b_hh1, W_nt, b_nt, W_cp, b_cp):
    src = edge_index[0]
    dst = edge_index[1]
    npad = _EP - _E
    src2d = jnp.concatenate(
        [src, jnp.zeros((npad,), src.dtype)]).reshape(_ROWS, _SUB)
    dst2d = jnp.concatenate(
        [dst, jnp.full((npad,), _N, dst.dtype)]).reshape(_ROWS, _SUB)

    degp = _deg_call(dst2d)
    d0 = degp[0, 0, :_N].reshape(_N, 1)
    d1 = degp[1, 0, :_N].reshape(_N, 1)

    p1, dinv = _mm1_call(x, d0, d1, gcn1_W)
    agg1 = _scat_call(p1, src2d, dst2d)
    p2 = _mm2_call(agg1, p1, dinv, gcn1_b.reshape(1, _GH), gcn2_W)
    agg2 = _scat_call(p2, src2d, dst2d)
    sums, cnts = _pool_call(agg2, p2, dinv, gcn2_b.reshape(1, _GH),
                            ln_w.reshape(1, _GH), ln_b.reshape(1, _GH),
                            batch.reshape(_GRID, 1, _BN))
    h1, h2, logits, probs = _head_call(
        text_embedding, sums, cnts,
        W_ih0, b_ih0.reshape(1, -1), b_hh0.reshape(1, -1),
        W_ih1, b_ih1.reshape(1, -1), b_hh1.reshape(1, -1),
        W_nt, b_nt.reshape(1, -1), W_cp, b_cp.reshape(1, -1))
    return logits, probs, jnp.stack([h1, h2], axis=0)


# speedup vs baseline: 26.8033x; 1.0016x over previous
import functools

import jax
import jax.numpy as jnp
from jax import lax
from jax.experimental import pallas as pl
from jax.experimental.pallas import tpu as pltpu
from jax.experimental.pallas import tpu_sc as plsc

_N = 50000
_E = 800000
_B = 256
_NT = 74
_GH = 64
_SH = 128
_MN = 100

_SUB = 128
_EP = 819200
_ROWS = _EP // _SUB
_RPT = _ROWS // 16
_STAGE = 16
_WIN = 64
_CB = 34
_DEG_RPT = _ROWS // 32
_DEG_PAD = 51200
_HN = _N // 2
_SPROWS = 25216
_DUMP = 25088

_BN = 10000
_GRID = _N // _BN

@functools.lru_cache(maxsize=None)
def _sc_mesh():
    return plsc.VectorSubcoreMesh(core_axis_name="c", subcore_axis_name="s",
                                  num_cores=2, num_subcores=16)


def _deg_body(dst_hbm, out_hbm, zbuf, idxbuf, onesbuf, deg_sp):
    c = lax.axis_index("c")
    s = lax.axis_index("s")

    def _zero(i, _):
        zbuf[pl.ds(i * 16, 16)] = jnp.zeros((16,), jnp.float32)
        return _
    lax.fori_loop(0, 3200 // 16, _zero, None)
    pltpu.sync_copy(zbuf, deg_sp.at[pl.ds(s * 3200, 3200)])

    def _ones(i, _):
        onesbuf[pl.ds(i * 16, 16)] = jnp.ones((16,), jnp.float32)
        return _
    lax.fori_loop(0, 8, _ones, None)
    plsc.subcore_barrier()

    w = s * 2 + c
    pltpu.sync_copy(dst_hbm.at[pl.ds(w * _DEG_RPT, _DEG_RPT)], idxbuf)

    def _scat(i, _):
        pltpu.sync_copy(onesbuf, deg_sp.at[idxbuf.at[i]], add=True)
        return _
    lax.fori_loop(0, _DEG_RPT, _scat, None)
    plsc.subcore_barrier()
    pltpu.sync_copy(deg_sp.at[pl.ds(s * 3200, 3200)],
                    out_hbm.at[c, 0, pl.ds(s * 3200, 3200)])


@functools.lru_cache(maxsize=None)
def _deg_kernel():
    return pl.kernel(
        _deg_body,
        out_type=jax.ShapeDtypeStruct((2, 1, _DEG_PAD), jnp.float32),
        mesh=_sc_mesh(),
        scratch_types=[
            pltpu.VMEM((3200,), jnp.float32),
            pltpu.VMEM((_DEG_RPT, _SUB), jnp.int32),
            pltpu.VMEM((_SUB,), jnp.float32),
            pltpu.VMEM_SHARED((_DEG_PAD,), jnp.float32),
        ],
        compiler_params=pltpu.CompilerParams(use_tc_tiling_on_sc=False, needs_layout_passes=False),
    )


def _scat_body(p_hbm, src_hbm, dst_hbm, out_hbm,
               sbA, dbA, sbB, dbB, csrc, cdst, r0, r1, r2, agg_sp,
               m0, m1, m2, msA, msB):
    c = lax.axis_index("c")
    s = lax.axis_index("s")
    base = c * _HN

    def _zero(i, _):
        r0[i // 4, pl.ds((i % 4) * 16, 16)] = jnp.zeros((16,), jnp.float32)
        return _
    lax.fori_loop(0, 256, _zero, None)

    def _zs(i, _):
        pltpu.sync_copy(r0, agg_sp.at[pl.ds(s * 1576 + i * 64, 64)])
        return _
    lax.fori_loop(0, 24, _zs, None)
    pltpu.sync_copy(r0.at[pl.ds(0, 40)],
                    agg_sp.at[pl.ds(s * 1576 + 1536, 40)])
    plsc.subcore_barrier()

    def _one(j, rbuf, sem):
        pltpu.async_copy(p_hbm.at[csrc.at[j]], rbuf, sem).wait()
        pltpu.sync_copy(rbuf, agg_sp.at[cdst.at[j]], add=True)

    def _flush(nw):
        def _tri(q, _):
            j0 = 3 * q
            d0 = pltpu.async_copy(p_hbm.at[csrc.at[j0]], r0, m0)
            d1 = pltpu.async_copy(p_hbm.at[csrc.at[j0 + 1]], r1, m1)
            d2 = pltpu.async_copy(p_hbm.at[csrc.at[j0 + 2]], r2, m2)
            d0.wait()
            pltpu.sync_copy(r0, agg_sp.at[cdst.at[j0]], add=True)
            d1.wait()
            pltpu.sync_copy(r1, agg_sp.at[cdst.at[j0 + 1]], add=True)
            d2.wait()
            pltpu.sync_copy(r2, agg_sp.at[cdst.at[j0 + 2]], add=True)
            return _
        nt = nw // 3
        lax.fori_loop(0, nt, _tri, None)

        def _rem(j, _):
            _one(j, r0, m0)
            return _
        lax.fori_loop(nt * 3, nw, _rem, None)

    def _append(i, off, sbuf, dbuf):
        def _app(j, off):
            sv = sbuf[i, pl.ds(j * 16, 16)]
            dv = dbuf[i, pl.ds(j * 16, 16)]
            m = (dv >= base) & (dv < base + _HN)
            mi = m.astype(jnp.int32)
            pos = off + plsc.cumsum(mi) - mi
            row = pos >> 6
            col = pos & 63
            plsc.store_scatter(csrc, [row, col], sv, mask=m)
            plsc.store_scatter(cdst, [row, col], dv - base, mask=m)
            return off + jnp.sum(mi)
        return lax.fori_loop(0, _SUB // 16, _app, off)

    def _carry(nw):
        def _mv(j, _):
            csrc[0, pl.ds(j * 16, 16)] = csrc[nw, pl.ds(j * 16, 16)]
            cdst[0, pl.ds(j * 16, 16)] = cdst[nw, pl.ds(j * 16, 16)]
            return _
        lax.fori_loop(0, 4, _mv, None)

    def _process(off, sbuf, dbuf):
        def _inner(i, off):
            return _append(i, off, sbuf, dbuf)
        off = lax.fori_loop(0, _STAGE, _inner, off)
        nw = off >> 6
        _flush(nw)
        _carry(nw)
        return off & 63

    def _pair(q, off):
        rb = s * _RPT + 2 * q * _STAGE
        dA0 = pltpu.async_copy(src_hbm.at[pl.ds(rb, _STAGE)], sbA, msA)
        dA1 = pltpu.async_copy(dst_hbm.at[pl.ds(rb, _STAGE)], dbA, msA)
        dB0 = pltpu.async_copy(
            src_hbm.at[pl.ds(rb + _STAGE, _STAGE)], sbB, msB)
        dB1 = pltpu.async_copy(
            dst_hbm.at[pl.ds(rb + _STAGE, _STAGE)], dbB, msB)
        dA0.wait()
        dA1.wait()
        off = _process(off, sbA, dbA)
        dB0.wait()
        dB1.wait()
        return _process(off, sbB, dbB)

    off = lax.fori_loop(0, 12, _pair, jnp.int32(0))
    rb_last = s * _RPT + 24 * _STAGE
    dL0 = pltpu.async_copy(src_hbm.at[pl.ds(rb_last, _STAGE)], sbA, msA)
    dL1 = pltpu.async_copy(dst_hbm.at[pl.ds(rb_last, _STAGE)], dbA, msA)
    dL0.wait()
    dL1.wait()
    off = _process(off, sbA, dbA)

    @pl.when(off > 0)
    def _tail_win():
        zrow = jnp.zeros((16,), jnp.int32)
        def _pad(j, _):
            lane = j * 16 + lax.iota(jnp.int32, 16)
            mp = lane >= off
            plsc.store_scatter(csrc, [zrow, lane], zrow, mask=mp)
            plsc.store_scatter(cdst, [zrow, lane], _DUMP + lane, mask=mp)
            return _
        lax.fori_loop(0, 4, _pad, None)
        _one(0, r0, m0)

    plsc.subcore_barrier()

    pltpu.sync_copy(agg_sp.at[pl.ds(s * 1560, 1560)],
                    out_hbm.at[pl.ds(base + s * 1560, 1560)])

    @pl.when(s < 5)
    def _tail():
        pltpu.sync_copy(agg_sp.at[pl.ds(24960 + s * 8, 8)],
                        out_hbm.at[pl.ds(base + 24960 + s * 8, 8)])


@functools.lru_cache(maxsize=None)
def _scat_kernel():
    return pl.kernel(
        _scat_body,
        out_type=jax.ShapeDtypeStruct((_N, _GH), jnp.float32),
        mesh=_sc_mesh(),
        scratch_types=[
            pltpu.VMEM((_STAGE, _SUB), jnp.int32),
            pltpu.VMEM((_STAGE, _SUB), jnp.int32),
            pltpu.VMEM((_STAGE, _SUB), jnp.int32),
            pltpu.VMEM((_STAGE, _SUB), jnp.int32),
            pltpu.VMEM((_CB, _WIN), jnp.int32),
            pltpu.VMEM((_CB, _WIN), jnp.int32),
            pltpu.VMEM((_WIN, _GH), jnp.float32),
            pltpu.VMEM((_WIN, _GH), jnp.float32),
            pltpu.VMEM((_WIN, _GH), jnp.float32),
            pltpu.VMEM_SHARED((_SPROWS, _GH), jnp.float32),
            pltpu.SemaphoreType.DMA,
            pltpu.SemaphoreType.DMA,
            pltpu.SemaphoreType.DMA,
            pltpu.SemaphoreType.DMA,
            pltpu.SemaphoreType.DMA,
        ],
        compiler_params=pltpu.CompilerParams(use_tc_tiling_on_sc=False, needs_layout_passes=False),
    )


def _mm1_body(x_ref, d0_ref, d1_ref, w_ref, p_ref, dinv_ref):
    deg = d0_ref[...] + d1_ref[...] + 1.0
    dinv = lax.rsqrt(deg)
    h = jnp.dot(x_ref[...], w_ref[...], preferred_element_type=jnp.float32)
    p_ref[...] = h * dinv
    dinv_ref[...] = dinv


def _mm1_call(x, d0, d1, w1):
    return pl.pallas_call(
        _mm1_body,
        grid=(_GRID,),
        in_specs=[
            pl.BlockSpec((_BN, _NT), lambda i: (i, 0)),
            pl.BlockSpec((_BN, 1), lambda i: (i, 0)),
            pl.BlockSpec((_BN, 1), lambda i: (i, 0)),
            pl.BlockSpec((_NT, _GH), lambda i: (0, 0)),
        ],
        out_specs=[
            pl.BlockSpec((_BN, _GH), lambda i: (i, 0)),
            pl.BlockSpec((_BN, 1), lambda i: (i, 0)),
        ],
        out_shape=[
            jax.ShapeDtypeStruct((_N, _GH), jnp.float32),
            jax.ShapeDtypeStruct((_N, 1), jnp.float32),
        ],
    )(x, d0, d1, w1)


def _mm2_body(agg_ref, p_ref, dinv_ref, b1_ref, w2_ref, p2_ref):
    dinv = dinv_ref[...]
    out1 = jnp.maximum(
        dinv * (agg_ref[...] + p_ref[...]) + b1_ref[...], 0.0)
    p2_ref[...] = jnp.dot(
        out1, w2_ref[...], preferred_element_type=jnp.float32) * dinv


def _mm2_call(agg1, p1, dinv, b1, w2):
    return pl.pallas_call(
        _mm2_body,
        grid=(_GRID,),
        in_specs=[
            pl.BlockSpec((_BN, _GH), lambda i: (i, 0)),
            pl.BlockSpec((_BN, _GH), lambda i: (i, 0)),
            pl.BlockSpec((_BN, 1), lambda i: (i, 0)),
            pl.BlockSpec((1, _GH), lambda i: (0, 0)),
            pl.BlockSpec((_GH, _GH), lambda i: (0, 0)),
        ],
        out_specs=pl.BlockSpec((_BN, _GH), lambda i: (i, 0)),
        out_shape=jax.ShapeDtypeStruct((_N, _GH), jnp.float32),
    )(agg1, p1, dinv, b1, w2)


def _pool_body(agg_ref, p_ref, dinv_ref, b2_ref, lnw_ref, lnb_ref, batch_ref,
               sum_ref, cnt_ref):
    i = pl.program_id(0)
    h = dinv_ref[...] * (agg_ref[...] + p_ref[...]) + b2_ref[...]
    mu = jnp.mean(h, axis=-1, keepdims=True)
    var = jnp.mean((h - mu) ** 2, axis=-1, keepdims=True)
    hn = (h - mu) / jnp.sqrt(var + 1e-5) * lnw_ref[...] + lnb_ref[...]
    oh = (lax.broadcasted_iota(jnp.int32, (_B, _BN), 0)
          == batch_ref[...].reshape(1, _BN)).astype(jnp.float32)
    ps = lax.dot_general(oh, hn, (((1,), (0,)), ((), ())),
                         preferred_element_type=jnp.float32)
    pc = jnp.sum(oh, axis=1, keepdims=True)

    @pl.when(i == 0)
    def _():
        sum_ref[...] = jnp.zeros_like(sum_ref)
        cnt_ref[...] = jnp.zeros_like(cnt_ref)

    sum_ref[...] += ps
    cnt_ref[...] += pc


def _pool_call(agg2, p2, dinv, b2, lnw, lnb, batch2d):
    return pl.pallas_call(
        _pool_body,
        grid=(_GRID,),
        in_specs=[
            pl.BlockSpec((_BN, _GH), lambda i: (i, 0)),
            pl.BlockSpec((_BN, _GH), lambda i: (i, 0)),
            pl.BlockSpec((_BN, 1), lambda i: (i, 0)),
            pl.BlockSpec((1, _GH), lambda i: (0, 0)),
            pl.BlockSpec((1, _GH), lambda i: (0, 0)),
            pl.BlockSpec((1, _GH), lambda i: (0, 0)),
            pl.BlockSpec((1, 1, _BN), lambda i: (i, 0, 0)),
        ],
        out_specs=[
            pl.BlockSpec((_B, _GH), lambda i: (0, 0)),
            pl.BlockSpec((_B, 1), lambda i: (0, 0)),
        ],
        out_shape=[
            jax.ShapeDtypeStruct((_B, _GH), jnp.float32),
            jax.ShapeDtypeStruct((_B, 1), jnp.float32),
        ],
    )(agg2, p2, dinv, b2, lnw, lnb, batch2d)


def _head_body(text_ref, sum_ref, cnt_ref, wi0_ref, bi0_ref, bh0_ref,
               wi1_ref, bi1_ref, bh1_ref, wnt_ref, bnt_ref, wcp_ref, bcp_ref,
               h1_ref, h2_ref, lg_ref, pr_ref):
    cnt = jnp.maximum(cnt_ref[...], 1.0)
    mean = sum_ref[...] / cnt
    comb = jnp.concatenate([text_ref[...], mean], axis=1)

    def _cell(xv, wih, bih, bhh):
        gi = lax.dot_general(xv, wih, (((1,), (1,)), ((), ())),
                             preferred_element_type=jnp.float32) + bih
        r = jax.nn.sigmoid(gi[:, :_SH] + bhh[:, :_SH])
        z = jax.nn.sigmoid(gi[:, _SH:2 * _SH] + bhh[:, _SH:2 * _SH])
        cg = jnp.tanh(gi[:, 2 * _SH:] + r * bhh[:, 2 * _SH:])
        return (1.0 - z) * cg

    h1 = _cell(comb, wi0_ref[...], bi0_ref[...], bh0_ref[...])
    h2 = _cell(h1, wi1_ref[...], bi1_ref[...], bh1_ref[...])
    h1_ref[...] = h1
    h2_ref[...] = h2
    lg_ref[...] = lax.dot_general(
        h2, wnt_ref[...], (((1,), (1,)), ((), ())),
        preferred_element_type=jnp.float32) + bnt_ref[...]
    pr_ref[...] = jax.nn.sigmoid(lax.dot_general(
        h2, wcp_ref[...], (((1,), (1,)), ((), ())),
        preferred_element_type=jnp.float32) + bcp_ref[...])


def _head_call(text, sums, cnts, wi0, bi0, bh0, wi1, bi1, bh1,
               wnt, bnt, wcp, bcp):
    return pl.pallas_call(
        _head_body,
        out_shape=[
            jax.ShapeDtypeStruct((_B, _SH), jnp.float32),
            jax.ShapeDtypeStruct((_B, _SH), jnp.float32),
            jax.ShapeDtypeStruct((_B, _NT), jnp.float32),
            jax.ShapeDtypeStruct((_B, _MN), jnp.float32),
        ],
    )(text, sums, cnts, wi0, bi0, bh0, wi1, bi1, bh1, wnt, bnt, wcp, bcp)


def _deg_call(dst_deg2d):
    return _deg_kernel()(dst_deg2d)


def _scat_call(p, src2d, dst2d):
    return _scat_kernel()(p, src2d, dst2d)


def kernel(text_embedding, x, edge_index, batch, gcn1_W, gcn1_b, gcn2_W,
           gcn2_b, ln_w, ln_b, W_ih0, W_hh0, b_ih0, b_hh0, W_ih1, W_hh1,
           b_ih1, b_hh1, W_nt, b_nt, W_cp, b_cp):
    src = edge_index[0]
    dst = edge_index[1]
    npad = _EP - _E
    src2d = jnp.concatenate(
        [src, jnp.zeros((npad,), src.dtype)]).reshape(_ROWS, _SUB)
    dst2d = jnp.concatenate(
        [dst, jnp.full((npad,), _N, dst.dtype)]).reshape(_ROWS, _SUB)

    degp = _deg_call(dst2d)
    d0 = degp[0, 0, :_N].reshape(_N, 1)
    d1 = degp[1, 0, :_N].reshape(_N, 1)

    p1, dinv = _mm1_call(x, d0, d1, gcn1_W)
    agg1 = _scat_call(p1, src2d, dst2d)
    p2 = _mm2_call(agg1, p1, dinv, gcn1_b.reshape(1, _GH), gcn2_W)
    agg2 = _scat_call(p2, src2d, dst2d)
    sums, cnts = _pool_call(agg2, p2, dinv, gcn2_b.reshape(1, _GH),
                            ln_w.reshape(1, _GH), ln_b.reshape(1, _GH),
                            batch.reshape(_GRID, 1, _BN))
    h1, h2, logits, probs = _head_call(
        text_embedding, sums, cnts,
        W_ih0, b_ih0.reshape(1, -1), b_hh0.reshape(1, -1),
        W_ih1, b_ih1.reshape(1, -1), b_hh1.reshape(1, -1),
        W_nt, b_nt.reshape(1, -1), W_cp, b_cp.reshape(1, -1))
    return logits, probs, jnp.stack([h1, h2], axis=0)


# speedup vs baseline: 28.4818x; 1.0626x over previous
import functools

import jax
import jax.numpy as jnp
from jax import lax
from jax.experimental import pallas as pl
from jax.experimental.pallas import tpu as pltpu
from jax.experimental.pallas import tpu_sc as plsc

_N = 50000
_E = 800000
_B = 256
_NT = 74
_GH = 64
_SH = 128
_MN = 100

_SUB = 128
_EROWS = _E // _SUB
_RPT = 390
_STAGE = 15
_WIN = 64
_CB = 32
_DEG_RPT = 195
_DEG_PAD = 51200
_HN = _N // 2
_SPROWS = 25216
_DUMP = 25088

_BN = 10000
_GRID = _N // _BN

@functools.lru_cache(maxsize=None)
def _sc_mesh():
    return plsc.VectorSubcoreMesh(core_axis_name="c", subcore_axis_name="s",
                                  num_cores=2, num_subcores=16)


def _deg_body(dst_hbm, out_hbm, zbuf, idxbuf, onesbuf, deg_sp):
    c = lax.axis_index("c")
    s = lax.axis_index("s")

    def _zero(i, _):
        zbuf[pl.ds(i * 16, 16)] = jnp.zeros((16,), jnp.float32)
        return _
    lax.fori_loop(0, 3200 // 16, _zero, None)
    pltpu.sync_copy(zbuf, deg_sp.at[pl.ds(s * 3200, 3200)])

    def _ones(i, _):
        onesbuf[pl.ds(i * 16, 16)] = jnp.ones((16,), jnp.float32)
        return _
    lax.fori_loop(0, 8, _ones, None)
    plsc.subcore_barrier()

    w = s * 2 + c
    pltpu.sync_copy(dst_hbm.at[1, pl.ds(w * _DEG_RPT, _DEG_RPT)],
                    idxbuf.at[pl.ds(0, _DEG_RPT)])

    @pl.when(w < 10)
    def _tail_stage():
        pltpu.sync_copy(dst_hbm.at[1, pl.ds(32 * _DEG_RPT + w, 1)],
                        idxbuf.at[pl.ds(_DEG_RPT, 1)])

    def _scat(i, _):
        pltpu.sync_copy(onesbuf, deg_sp.at[idxbuf.at[i]], add=True)
        return _
    nrows = jnp.where(w < 10, _DEG_RPT + 1, _DEG_RPT)
    lax.fori_loop(0, nrows, _scat, None)
    plsc.subcore_barrier()
    pltpu.sync_copy(deg_sp.at[pl.ds(s * 3200, 3200)],
                    out_hbm.at[c, 0, pl.ds(s * 3200, 3200)])


@functools.lru_cache(maxsize=None)
def _deg_kernel():
    return pl.kernel(
        _deg_body,
        out_type=jax.ShapeDtypeStruct((2, 1, _DEG_PAD), jnp.float32),
        mesh=_sc_mesh(),
        scratch_types=[
            pltpu.VMEM((3200,), jnp.float32),
            pltpu.VMEM((_DEG_RPT + 1, _SUB), jnp.int32),
            pltpu.VMEM((_SUB,), jnp.float32),
            pltpu.VMEM_SHARED((_DEG_PAD,), jnp.float32),
        ],
        compiler_params=pltpu.CompilerParams(use_tc_tiling_on_sc=False, needs_layout_passes=False),
    )


def _scat_body(p_hbm, ei_hbm, out_hbm,
               sbA, dbA, sbB, dbB, csrc, cdst, r0, r1, r2, agg_sp,
               m0, m1, m2, msA, msB):
    c = lax.axis_index("c")
    s = lax.axis_index("s")
    base = c * _HN

    def _zero(i, _):
        r0[i // 4, pl.ds((i % 4) * 16, 16)] = jnp.zeros((16,), jnp.float32)
        return _
    lax.fori_loop(0, 256, _zero, None)

    def _zs(i, _):
        pltpu.sync_copy(r0, agg_sp.at[pl.ds(s * 1576 + i * 64, 64)])
        return _
    lax.fori_loop(0, 24, _zs, None)
    pltpu.sync_copy(r0.at[pl.ds(0, 40)],
                    agg_sp.at[pl.ds(s * 1576 + 1536, 40)])
    plsc.subcore_barrier()

    def _one(j, rbuf, sem):
        pltpu.async_copy(p_hbm.at[csrc.at[j]], rbuf, sem).wait()
        pltpu.sync_copy(rbuf, agg_sp.at[cdst.at[j]], add=True)

    def _flush(nw):
        def _tri(q, _):
            j0 = 3 * q
            d0 = pltpu.async_copy(p_hbm.at[csrc.at[j0]], r0, m0)
            d1 = pltpu.async_copy(p_hbm.at[csrc.at[j0 + 1]], r1, m1)
            d2 = pltpu.async_copy(p_hbm.at[csrc.at[j0 + 2]], r2, m2)
            d0.wait()
            pltpu.sync_copy(r0, agg_sp.at[cdst.at[j0]], add=True)
            d1.wait()
            pltpu.sync_copy(r1, agg_sp.at[cdst.at[j0 + 1]], add=True)
            d2.wait()
            pltpu.sync_copy(r2, agg_sp.at[cdst.at[j0 + 2]], add=True)
            return _
        nt = nw // 3
        lax.fori_loop(0, nt, _tri, None)

        def _rem(j, _):
            _one(j, r0, m0)
            return _
        lax.fori_loop(nt * 3, nw, _rem, None)

    def _append(i, off, sbuf, dbuf):
        def _app(j, off):
            sv = sbuf[i, pl.ds(j * 16, 16)]
            dv = dbuf[i, pl.ds(j * 16, 16)]
            m = (dv >= base) & (dv < base + _HN)
            mi = m.astype(jnp.int32)
            pos = off + plsc.cumsum(mi) - mi
            row = pos >> 6
            col = pos & 63
            plsc.store_scatter(csrc, [row, col], sv, mask=m)
            plsc.store_scatter(cdst, [row, col], dv - base, mask=m)
            return off + jnp.sum(mi)
        return lax.fori_loop(0, _SUB // 16, _app, off)

    def _carry(nw):
        def _mv(j, _):
            csrc[0, pl.ds(j * 16, 16)] = csrc[nw, pl.ds(j * 16, 16)]
            cdst[0, pl.ds(j * 16, 16)] = cdst[nw, pl.ds(j * 16, 16)]
            return _
        lax.fori_loop(0, 4, _mv, None)

    def _process(off, sbuf, dbuf, nrows):
        def _inner(i, off):
            return _append(i, off, sbuf, dbuf)
        off = lax.fori_loop(0, nrows, _inner, off)
        nw = off >> 6
        _flush(nw)
        _carry(nw)
        return off & 63

    def _pair(q, off):
        rb = s * _RPT + 2 * q * _STAGE
        dA0 = pltpu.async_copy(ei_hbm.at[0, pl.ds(rb, _STAGE)], sbA, msA)
        dA1 = pltpu.async_copy(ei_hbm.at[1, pl.ds(rb, _STAGE)], dbA, msA)
        dB0 = pltpu.async_copy(
            ei_hbm.at[0, pl.ds(rb + _STAGE, _STAGE)], sbB, msB)
        dB1 = pltpu.async_copy(
            ei_hbm.at[1, pl.ds(rb + _STAGE, _STAGE)], dbB, msB)
        dA0.wait()
        dA1.wait()
        off = _process(off, sbA, dbA, _STAGE)
        dB0.wait()
        dB1.wait()
        return _process(off, sbB, dbB, _STAGE)

    off = lax.fori_loop(0, 13, _pair, jnp.int32(0))

    @pl.when(s < 10)
    def _extra_row():
        r = 16 * _RPT + s
        pltpu.sync_copy(ei_hbm.at[0, pl.ds(r, 1)], sbA.at[pl.ds(0, 1)])
        pltpu.sync_copy(ei_hbm.at[1, pl.ds(r, 1)], dbA.at[pl.ds(0, 1)])

    @pl.when(s >= 10)
    def _fake_row():
        def _f(j, _):
            sbA[0, pl.ds(j * 16, 16)] = jnp.zeros((16,), jnp.int32)
            dbA[0, pl.ds(j * 16, 16)] = jnp.full((16,), _N, jnp.int32)
            return _
        lax.fori_loop(0, 8, _f, None)

    off = _process(off, sbA, dbA, 1)

    @pl.when(off > 0)
    def _tail_win():
        zrow = jnp.zeros((16,), jnp.int32)
        def _pad(j, _):
            lane = j * 16 + lax.iota(jnp.int32, 16)
            mp = lane >= off
            plsc.store_scatter(csrc, [zrow, lane], zrow, mask=mp)
            plsc.store_scatter(cdst, [zrow, lane], _DUMP + lane, mask=mp)
            return _
        lax.fori_loop(0, 4, _pad, None)
        _one(0, r0, m0)

    plsc.subcore_barrier()

    pltpu.sync_copy(agg_sp.at[pl.ds(s * 1560, 1560)],
                    out_hbm.at[pl.ds(base + s * 1560, 1560)])

    @pl.when(s < 5)
    def _tail():
        pltpu.sync_copy(agg_sp.at[pl.ds(24960 + s * 8, 8)],
                        out_hbm.at[pl.ds(base + 24960 + s * 8, 8)])


@functools.lru_cache(maxsize=None)
def _scat_kernel():
    return pl.kernel(
        _scat_body,
        out_type=jax.ShapeDtypeStruct((_N, _GH), jnp.float32),
        mesh=_sc_mesh(),
        scratch_types=[
            pltpu.VMEM((_STAGE, _SUB), jnp.int32),
            pltpu.VMEM((_STAGE, _SUB), jnp.int32),
            pltpu.VMEM((_STAGE, _SUB), jnp.int32),
            pltpu.VMEM((_STAGE, _SUB), jnp.int32),
            pltpu.VMEM((_CB, _WIN), jnp.int32),
            pltpu.VMEM((_CB, _WIN), jnp.int32),
            pltpu.VMEM((_WIN, _GH), jnp.float32),
            pltpu.VMEM((_WIN, _GH), jnp.float32),
            pltpu.VMEM((_WIN, _GH), jnp.float32),
            pltpu.VMEM_SHARED((_SPROWS, _GH), jnp.float32),
            pltpu.SemaphoreType.DMA,
            pltpu.SemaphoreType.DMA,
            pltpu.SemaphoreType.DMA,
            pltpu.SemaphoreType.DMA,
            pltpu.SemaphoreType.DMA,
        ],
        compiler_params=pltpu.CompilerParams(use_tc_tiling_on_sc=False, needs_layout_passes=False),
    )


def _mm1_body(x_ref, d0_ref, d1_ref, w_ref, p_ref, dinv_ref):
    deg = d0_ref[...] + d1_ref[...] + 1.0
    dinv = lax.rsqrt(deg)
    h = jnp.dot(x_ref[...], w_ref[...], preferred_element_type=jnp.float32)
    p_ref[...] = h * dinv
    dinv_ref[...] = dinv


def _mm1_call(x, d0, d1, w1):
    return pl.pallas_call(
        _mm1_body,
        grid=(_GRID,),
        in_specs=[
            pl.BlockSpec((_BN, _NT), lambda i: (i, 0)),
            pl.BlockSpec((_BN, 1), lambda i: (i, 0)),
            pl.BlockSpec((_BN, 1), lambda i: (i, 0)),
            pl.BlockSpec((_NT, _GH), lambda i: (0, 0)),
        ],
        out_specs=[
            pl.BlockSpec((_BN, _GH), lambda i: (i, 0)),
            pl.BlockSpec((_BN, 1), lambda i: (i, 0)),
        ],
        out_shape=[
            jax.ShapeDtypeStruct((_N, _GH), jnp.float32),
            jax.ShapeDtypeStruct((_N, 1), jnp.float32),
        ],
    )(x, d0, d1, w1)


def _mm2_body(agg_ref, p_ref, dinv_ref, b1_ref, w2_ref, p2_ref):
    dinv = dinv_ref[...]
    out1 = jnp.maximum(
        dinv * (agg_ref[...] + p_ref[...]) + b1_ref[...], 0.0)
    p2_ref[...] = jnp.dot(
        out1, w2_ref[...], preferred_element_type=jnp.float32) * dinv


def _mm2_call(agg1, p1, dinv, b1, w2):
    return pl.pallas_call(
        _mm2_body,
        grid=(_GRID,),
        in_specs=[
            pl.BlockSpec((_BN, _GH), lambda i: (i, 0)),
            pl.BlockSpec((_BN, _GH), lambda i: (i, 0)),
            pl.BlockSpec((_BN, 1), lambda i: (i, 0)),
            pl.BlockSpec((1, _GH), lambda i: (0, 0)),
            pl.BlockSpec((_GH, _GH), lambda i: (0, 0)),
        ],
        out_specs=pl.BlockSpec((_BN, _GH), lambda i: (i, 0)),
        out_shape=jax.ShapeDtypeStruct((_N, _GH), jnp.float32),
    )(agg1, p1, dinv, b1, w2)


def _pool_body(agg_ref, p_ref, dinv_ref, b2_ref, lnw_ref, lnb_ref, batch_ref,
               sum_ref, cnt_ref):
    i = pl.program_id(0)
    h = dinv_ref[...] * (agg_ref[...] + p_ref[...]) + b2_ref[...]
    mu = jnp.mean(h, axis=-1, keepdims=True)
    var = jnp.mean((h - mu) ** 2, axis=-1, keepdims=True)
    hn = (h - mu) / jnp.sqrt(var + 1e-5) * lnw_ref[...] + lnb_ref[...]
    oh = (lax.broadcasted_iota(jnp.int32, (_B, _BN), 0)
          == batch_ref[...].reshape(1, _BN)).astype(jnp.float32)
    ps = lax.dot_general(oh, hn, (((1,), (0,)), ((), ())),
                         preferred_element_type=jnp.float32)
    pc = jnp.sum(oh, axis=1, keepdims=True)

    @pl.when(i == 0)
    def _():
        sum_ref[...] = jnp.zeros_like(sum_ref)
        cnt_ref[...] = jnp.zeros_like(cnt_ref)

    sum_ref[...] += ps
    cnt_ref[...] += pc


def _pool_call(agg2, p2, dinv, b2, lnw, lnb, batch2d):
    return pl.pallas_call(
        _pool_body,
        grid=(_GRID,),
        in_specs=[
            pl.BlockSpec((_BN, _GH), lambda i: (i, 0)),
            pl.BlockSpec((_BN, _GH), lambda i: (i, 0)),
            pl.BlockSpec((_BN, 1), lambda i: (i, 0)),
            pl.BlockSpec((1, _GH), lambda i: (0, 0)),
            pl.BlockSpec((1, _GH), lambda i: (0, 0)),
            pl.BlockSpec((1, _GH), lambda i: (0, 0)),
            pl.BlockSpec((1, 1, _BN), lambda i: (i, 0, 0)),
        ],
        out_specs=[
            pl.BlockSpec((_B, _GH), lambda i: (0, 0)),
            pl.BlockSpec((_B, 1), lambda i: (0, 0)),
        ],
        out_shape=[
            jax.ShapeDtypeStruct((_B, _GH), jnp.float32),
            jax.ShapeDtypeStruct((_B, 1), jnp.float32),
        ],
    )(agg2, p2, dinv, b2, lnw, lnb, batch2d)


def _head_body(text_ref, sum_ref, cnt_ref, wi0_ref, bi0_ref, bh0_ref,
               wi1_ref, bi1_ref, bh1_ref, wnt_ref, bnt_ref, wcp_ref, bcp_ref,
               h1_ref, h2_ref, lg_ref, pr_ref):
    cnt = jnp.maximum(cnt_ref[...], 1.0)
    mean = sum_ref[...] / cnt
    comb = jnp.concatenate([text_ref[...], mean], axis=1)

    def _cell(xv, wih, bih, bhh):
        gi = lax.dot_general(xv, wih, (((1,), (1,)), ((), ())),
                             preferred_element_type=jnp.float32) + bih
        r = jax.nn.sigmoid(gi[:, :_SH] + bhh[:, :_SH])
        z = jax.nn.sigmoid(gi[:, _SH:2 * _SH] + bhh[:, _SH:2 * _SH])
        cg = jnp.tanh(gi[:, 2 * _SH:] + r * bhh[:, 2 * _SH:])
        return (1.0 - z) * cg

    h1 = _cell(comb, wi0_ref[...], bi0_ref[...], bh0_ref[...])
    h2 = _cell(h1, wi1_ref[...], bi1_ref[...], bh1_ref[...])
    h1_ref[...] = h1
    h2_ref[...] = h2
    lg_ref[...] = lax.dot_general(
        h2, wnt_ref[...], (((1,), (1,)), ((), ())),
        preferred_element_type=jnp.float32) + bnt_ref[...]
    pr_ref[...] = jax.nn.sigmoid(lax.dot_general(
        h2, wcp_ref[...], (((1,), (1,)), ((), ())),
        preferred_element_type=jnp.float32) + bcp_ref[...])


def _head_call(text, sums, cnts, wi0, bi0, bh0, wi1, bi1, bh1,
               wnt, bnt, wcp, bcp):
    return pl.pallas_call(
        _head_body,
        out_shape=[
            jax.ShapeDtypeStruct((_B, _SH), jnp.float32),
            jax.ShapeDtypeStruct((_B, _SH), jnp.float32),
            jax.ShapeDtypeStruct((_B, _NT), jnp.float32),
            jax.ShapeDtypeStruct((_B, _MN), jnp.float32),
        ],
    )(text, sums, cnts, wi0, bi0, bh0, wi1, bi1, bh1, wnt, bnt, wcp, bcp)


def _deg_call(ei3):
    return _deg_kernel()(ei3)


def _scat_call(p, ei3):
    return _scat_kernel()(p, ei3)


def kernel(text_embedding, x, edge_index, batch, gcn1_W, gcn1_b, gcn2_W,
           gcn2_b, ln_w, ln_b, W_ih0, W_hh0, b_ih0, b_hh0, W_ih1, W_hh1,
           b_ih1, b_hh1, W_nt, b_nt, W_cp, b_cp):
    ei3 = edge_index.reshape(2, _EROWS, _SUB)

    degp = _deg_call(ei3)
    d0 = degp[0, 0, :_N].reshape(_N, 1)
    d1 = degp[1, 0, :_N].reshape(_N, 1)

    p1, dinv = _mm1_call(x, d0, d1, gcn1_W)
    agg1 = _scat_call(p1, ei3)
    p2 = _mm2_call(agg1, p1, dinv, gcn1_b.reshape(1, _GH), gcn2_W)
    agg2 = _scat_call(p2, ei3)
    sums, cnts = _pool_call(agg2, p2, dinv, gcn2_b.reshape(1, _GH),
                            ln_w.reshape(1, _GH), ln_b.reshape(1, _GH),
                            batch.reshape(_GRID, 1, _BN))
    h1, h2, logits, probs = _head_call(
        text_embedding, sums, cnts,
        W_ih0, b_ih0.reshape(1, -1), b_hh0.reshape(1, -1),
        W_ih1, b_ih1.reshape(1, -1), b_hh1.reshape(1, -1),
        W_nt, b_nt.reshape(1, -1), W_cp, b_cp.reshape(1, -1))
    return logits, probs, jnp.stack([h1, h2], axis=0)
